# Initial kernel scaffold; baseline (speedup 1.0000x reference)
import jax
import jax.numpy as jnp
from jax.experimental import pallas as pl


def kernel(nodes, edges, graph, node_W0, node_b0, node_W1, node_b1, edge_W0, edge_b0, edge_W1, edge_b1):
    raise NotImplementedError("write your pallas kernel here")


# speedup vs baseline: 2.8856x; 2.8856x over previous
import functools
import jax
import jax.numpy as jnp
from jax import lax
from jax.experimental import pallas as pl
from jax.experimental.pallas import tpu as pltpu
from jax.experimental.pallas import tpu_sc as plsc

N = 10000
E = 320000
D = 128

NC = 2
NS = 16
NW = NC * NS
EW = E // NW
CHUNK = 80
NCHUNK = EW // CHUNK
NACC = N // CHUNK

def _scatter_body(edges_hbm, dstc_hbm, out_hbm, idx_v, ebuf, acc):
    c = lax.axis_index("c")
    s = lax.axis_index("s")
    wid = c * NS + s

    @pl.loop(0, CHUNK)
    def _zrow(i):
        @pl.loop(0, D // 16, unroll=8)
        def _zlane(l):
            ebuf[i, pl.ds(l * 16, 16)] = jnp.zeros((16,), jnp.float32)

    @pl.loop(s, NACC, step=NS)
    def _zacc(k):
        pltpu.sync_copy(ebuf, acc.at[pl.ds(k * CHUNK, CHUNK)])

    plsc.subcore_barrier()

    pltpu.sync_copy(dstc_hbm.at[wid], idx_v)

    @pl.loop(0, NCHUNK)
    def _chunk(j):
        pltpu.sync_copy(edges_hbm.at[pl.ds(wid * EW + j * CHUNK, CHUNK)], ebuf)
        pltpu.sync_copy(ebuf, acc.at[idx_v.at[j]], add=True)

    plsc.subcore_barrier()

    @pl.loop(s, NACC, step=NS)
    def _wacc(k):
        pltpu.sync_copy(acc.at[pl.ds(k * CHUNK, CHUNK)],
                        out_hbm.at[c, pl.ds(k * CHUNK, CHUNK)])


def _gather_body(p_hbm, q_hbm, srcc_hbm, dstc_hbm, gp_hbm, gq_hbm,
                 isrc, idst, buf):
    c = lax.axis_index("c")
    s = lax.axis_index("s")
    wid = c * NS + s

    pltpu.sync_copy(srcc_hbm.at[wid], isrc)
    pltpu.sync_copy(dstc_hbm.at[wid], idst)

    @pl.loop(0, NCHUNK)
    def _chunk(j):
        base = wid * EW + j * CHUNK
        pltpu.sync_copy(p_hbm.at[isrc.at[j]], buf)
        pltpu.sync_copy(buf, gp_hbm.at[pl.ds(base, CHUNK)])
        pltpu.sync_copy(q_hbm.at[idst.at[j]], buf)
        pltpu.sync_copy(buf, gq_hbm.at[pl.ds(base, CHUNK)])


@functools.cache
def _sc_calls():
    mesh = plsc.VectorSubcoreMesh(
        core_axis_name="c", subcore_axis_name="s",
        num_cores=NC, num_subcores=NS)
    scatter_call = pl.kernel(
        _scatter_body,
        out_type=jax.ShapeDtypeStruct((NC, N, D), jnp.float32),
        mesh=mesh,
        scratch_types=[
            pltpu.VMEM((NCHUNK, CHUNK), jnp.int32),
            pltpu.VMEM((CHUNK, D), jnp.float32),
            pltpu.VMEM_SHARED((N, D), jnp.float32),
        ],
    )
    gather_call = pl.kernel(
        _gather_body,
        out_type=(jax.ShapeDtypeStruct((E, D), jnp.float32),
                  jax.ShapeDtypeStruct((E, D), jnp.float32)),
        mesh=mesh,
        scratch_types=[
            pltpu.VMEM((NCHUNK, CHUNK), jnp.int32),
            pltpu.VMEM((NCHUNK, CHUNK), jnp.int32),
            pltpu.VMEM((CHUNK, D), jnp.float32),
        ],
    )
    return scatter_call, gather_call


NODE_BLK = 2000
EDGE_BLK = 2000


def _node_body(nodes_ref, pr_ref, w0_ref, b0_ref, w1_ref, b1_ref,
               ew0a_ref, ew0b_ref, nn_ref, p_ref, q_ref):
    msg = pr_ref[0] + pr_ref[1]
    x = nodes_ref[...]
    h = jnp.maximum(
        jnp.dot(x, w0_ref[:D, :], preferred_element_type=jnp.float32)
        + jnp.dot(msg, w0_ref[D:, :], preferred_element_type=jnp.float32)
        + b0_ref[...], 0.0)
    nn = jnp.maximum(
        jnp.dot(h, w1_ref[...], preferred_element_type=jnp.float32)
        + b1_ref[...], 0.0) + x
    nn_ref[...] = nn
    p_ref[...] = jnp.dot(nn, ew0a_ref[...], preferred_element_type=jnp.float32)
    q_ref[...] = jnp.dot(nn, ew0b_ref[...], preferred_element_type=jnp.float32)


def _edge_body(gp_ref, gq_ref, edges_ref, ew0c_ref, eb0_ref, ew1_ref, eb1_ref,
               out_ref):
    e = edges_ref[...]
    he = jnp.maximum(
        gp_ref[...] + gq_ref[...]
        + jnp.dot(e, ew0c_ref[...], preferred_element_type=jnp.float32)
        + eb0_ref[...], 0.0)
    out_ref[...] = jnp.tanh(
        jnp.dot(he, ew1_ref[...], preferred_element_type=jnp.float32)
        + eb1_ref[...]) + e


def _full(shape):
    return pl.BlockSpec(shape, lambda i: (0,) * len(shape))


def kernel(nodes, edges, graph, node_W0, node_b0, node_W1, node_b1,
           edge_W0, edge_b0, edge_W1, edge_b1):
    srcc = graph[0].reshape(NW, NCHUNK, CHUNK)
    dstc = graph[1].reshape(NW, NCHUNK, CHUNK)

    scatter_call, gather_call = _sc_calls()
    partials = scatter_call(edges, dstc)

    new_nodes, p_tab, q_tab = pl.pallas_call(
        _node_body,
        grid=(N // NODE_BLK,),
        in_specs=[
            pl.BlockSpec((NODE_BLK, D), lambda i: (i, 0)),
            pl.BlockSpec((NC, NODE_BLK, D), lambda i: (0, i, 0)),
            _full((2 * D, D)),
            _full((1, D)),
            _full((D, D)),
            _full((1, D)),
            _full((D, D)),
            _full((D, D)),
        ],
        out_specs=[
            pl.BlockSpec((NODE_BLK, D), lambda i: (i, 0)),
            pl.BlockSpec((NODE_BLK, D), lambda i: (i, 0)),
            pl.BlockSpec((NODE_BLK, D), lambda i: (i, 0)),
        ],
        out_shape=[
            jax.ShapeDtypeStruct((N, D), jnp.float32),
            jax.ShapeDtypeStruct((N, D), jnp.float32),
            jax.ShapeDtypeStruct((N, D), jnp.float32),
        ],
    )(nodes, partials, node_W0, node_b0.reshape(1, D), node_W1,
      node_b1.reshape(1, D), edge_W0[:D, :], edge_W0[D:2 * D, :])

    gp, gq = gather_call(p_tab, q_tab, srcc, dstc)

    new_edges = pl.pallas_call(
        _edge_body,
        grid=(E // EDGE_BLK,),
        in_specs=[
            pl.BlockSpec((EDGE_BLK, D), lambda i: (i, 0)),
            pl.BlockSpec((EDGE_BLK, D), lambda i: (i, 0)),
            pl.BlockSpec((EDGE_BLK, D), lambda i: (i, 0)),
            _full((D, D)),
            _full((1, D)),
            _full((D, D)),
            _full((1, D)),
        ],
        out_specs=pl.BlockSpec((EDGE_BLK, D), lambda i: (i, 0)),
        out_shape=jax.ShapeDtypeStruct((E, D), jnp.float32),
    )(gp, gq, edges, edge_W0[2 * D:, :], edge_b0.reshape(1, D), edge_W1,
      edge_b1.reshape(1, D))

    return (new_nodes, new_edges)


# speedup vs baseline: 3.9564x; 1.3711x over previous
import functools
import jax
import jax.numpy as jnp
from jax import lax
from jax.experimental import pallas as pl
from jax.experimental.pallas import tpu as pltpu
from jax.experimental.pallas import tpu_sc as plsc

N = 10000
E = 320000
D = 128

NC = 2
NS = 16
NW = NC * NS
EW = E // NW

SCHUNK = 40
SNCH = EW // SCHUNK
NACC = N // SCHUNK
GCHUNK = 40
GNCH = EW // GCHUNK
NB = 5
NBS = 2


def _scatter_body(edges_hbm, dstc_hbm, out_hbm, idx_v, ebuf, acc,
                  sem_in, sem_sc):
    c = lax.axis_index("c")
    s = lax.axis_index("s")
    wid = c * NS + s
    ebase = wid * EW

    pltpu.sync_copy(dstc_hbm.at[wid], idx_v)

    @pl.loop(0, SCHUNK)
    def _zrow(i):
        @pl.loop(0, D // 16, unroll=8)
        def _zlane(l):
            ebuf[0, i, pl.ds(l * 16, 16)] = jnp.zeros((16,), jnp.float32)

    @pl.loop(s, NACC, step=NS)
    def _zacc(k):
        pltpu.sync_copy(ebuf.at[0], acc.at[pl.ds(k * SCHUNK, SCHUNK)])

    plsc.subcore_barrier()

    for b in range(NBS):
        pltpu.async_copy(edges_hbm.at[pl.ds(ebase + b * SCHUNK, SCHUNK)],
                         ebuf.at[b], sem_in.at[b])

    @pl.loop(0, SNCH // NBS)
    def _group(g):
        for b in range(NBS):
            k = g * NBS + b
            bp = (b - 1) % NBS
            pltpu.make_async_copy(
                edges_hbm.at[pl.ds(ebase + k * SCHUNK, SCHUNK)],
                ebuf.at[b], sem_in.at[b]).wait()
            pltpu.async_copy(ebuf.at[b], acc.at[idx_v.at[k]], sem_sc.at[b],
                             add=True)

            @pl.when(k >= 1)
            def _drain_prev():
                pltpu.make_async_copy(
                    ebuf.at[bp], acc.at[idx_v.at[k - 1]], sem_sc.at[bp]).wait()

                @pl.when(k + NBS - 1 < SNCH)
                def _refill():
                    pltpu.async_copy(
                        edges_hbm.at[pl.ds(ebase + (k + NBS - 1) * SCHUNK,
                                           SCHUNK)],
                        ebuf.at[bp], sem_in.at[bp])

    last = SNCH - 1
    pltpu.make_async_copy(ebuf.at[last % NBS], acc.at[idx_v.at[last]],
                          sem_sc.at[last % NBS]).wait()

    plsc.subcore_barrier()

    @pl.loop(s, NACC, step=NS)
    def _wacc(k):
        pltpu.sync_copy(acc.at[pl.ds(k * SCHUNK, SCHUNK)],
                        out_hbm.at[c, pl.ds(k * SCHUNK, SCHUNK)])


def _gather_body(p_hbm, q_hbm, srcc_hbm, dstc_hbm, g_hbm,
                 isrc, idst, pbuf, qbuf, sem_p, sem_q, sem_out):
    c = lax.axis_index("c")
    s = lax.axis_index("s")
    wid = c * NS + s
    ebase = wid * EW

    pltpu.sync_copy(srcc_hbm.at[wid], isrc)
    pltpu.sync_copy(dstc_hbm.at[wid], idst)

    for b in range(NB):
        pltpu.async_copy(p_hbm.at[isrc.at[b]], pbuf.at[b], sem_p.at[b])
        pltpu.async_copy(q_hbm.at[idst.at[b]], qbuf.at[b], sem_q.at[b])

    @pl.loop(0, GNCH // NB)
    def _group(g):
        for b in range(NB):
            k = g * NB + b
            bp = (b - 1) % NB
            pltpu.make_async_copy(p_hbm.at[isrc.at[k]], pbuf.at[b],
                                  sem_p.at[b]).wait()
            pltpu.make_async_copy(q_hbm.at[idst.at[k]], qbuf.at[b],
                                  sem_q.at[b]).wait()

            @pl.loop(0, GCHUNK)
            def _row(i):
                @pl.loop(0, D // 16, unroll=8)
                def _lane(l):
                    sl = pl.ds(l * 16, 16)
                    pbuf[b, i, sl] = pbuf[b, i, sl] + qbuf[b, i, sl]

            pltpu.async_copy(pbuf.at[b],
                             g_hbm.at[pl.ds(ebase + k * GCHUNK, GCHUNK)],
                             sem_out.at[b])

            @pl.when(k >= 1)
            def _drain_prev():
                pltpu.make_async_copy(
                    pbuf.at[bp],
                    g_hbm.at[pl.ds(ebase + (k - 1) * GCHUNK, GCHUNK)],
                    sem_out.at[bp]).wait()

                @pl.when(k + NB - 1 < GNCH)
                def _refill():
                    kk = k + NB - 1
                    pltpu.async_copy(p_hbm.at[isrc.at[kk]], pbuf.at[bp],
                                     sem_p.at[bp])
                    pltpu.async_copy(q_hbm.at[idst.at[kk]], qbuf.at[bp],
                                     sem_q.at[bp])

    last = GNCH - 1
    pltpu.make_async_copy(pbuf.at[last % NB],
                          g_hbm.at[pl.ds(ebase + last * GCHUNK, GCHUNK)],
                          sem_out.at[last % NB]).wait()


@functools.cache
def _sc_calls():
    mesh = plsc.VectorSubcoreMesh(
        core_axis_name="c", subcore_axis_name="s",
        num_cores=NC, num_subcores=NS)
    scatter_call = pl.kernel(
        _scatter_body,
        out_type=jax.ShapeDtypeStruct((NC, N, D), jnp.float32),
        mesh=mesh,
        scratch_types=[
            pltpu.VMEM((SNCH, SCHUNK), jnp.int32),
            pltpu.VMEM((NBS, SCHUNK, D), jnp.float32),
            pltpu.VMEM_SHARED((N, D), jnp.float32),
            pltpu.SemaphoreType.DMA((NBS,)),
            pltpu.SemaphoreType.DMA((NBS,)),
        ],
    )
    gather_call = pl.kernel(
        _gather_body,
        out_type=jax.ShapeDtypeStruct((E, D), jnp.float32),
        mesh=mesh,
        scratch_types=[
            pltpu.VMEM((GNCH, GCHUNK), jnp.int32),
            pltpu.VMEM((GNCH, GCHUNK), jnp.int32),
            pltpu.VMEM((NB, GCHUNK, D), jnp.float32),
            pltpu.VMEM((NB, GCHUNK, D), jnp.float32),
            pltpu.SemaphoreType.DMA((NB,)),
            pltpu.SemaphoreType.DMA((NB,)),
            pltpu.SemaphoreType.DMA((NB,)),
        ],
    )
    return scatter_call, gather_call


NODE_BLK = 2000
EDGE_BLK = 2000


def _node_body(nodes_ref, pr_ref, w0_ref, b0_ref, w1_ref, b1_ref,
               ew0a_ref, ew0b_ref, nn_ref, p_ref, q_ref):
    msg = pr_ref[0] + pr_ref[1]
    x = nodes_ref[...]
    h = jnp.maximum(
        jnp.dot(x, w0_ref[:D, :], preferred_element_type=jnp.float32)
        + jnp.dot(msg, w0_ref[D:, :], preferred_element_type=jnp.float32)
        + b0_ref[...], 0.0)
    nn = jnp.maximum(
        jnp.dot(h, w1_ref[...], preferred_element_type=jnp.float32)
        + b1_ref[...], 0.0) + x
    nn_ref[...] = nn
    p_ref[...] = jnp.dot(nn, ew0a_ref[...], preferred_element_type=jnp.float32)
    q_ref[...] = jnp.dot(nn, ew0b_ref[...], preferred_element_type=jnp.float32)


def _edge_body(g_ref, edges_ref, ew0c_ref, eb0_ref, ew1_ref, eb1_ref,
               out_ref):
    e = edges_ref[...]
    he = jnp.maximum(
        g_ref[...]
        + jnp.dot(e, ew0c_ref[...], preferred_element_type=jnp.float32)
        + eb0_ref[...], 0.0)
    out_ref[...] = jnp.tanh(
        jnp.dot(he, ew1_ref[...], preferred_element_type=jnp.float32)
        + eb1_ref[...]) + e


def _full(shape):
    return pl.BlockSpec(shape, lambda i: (0,) * len(shape))


def kernel(nodes, edges, graph, node_W0, node_b0, node_W1, node_b1,
           edge_W0, edge_b0, edge_W1, edge_b1):
    sdstc = graph[1].reshape(NW, SNCH, SCHUNK)
    gsrcc = graph[0].reshape(NW, GNCH, GCHUNK)
    gdstc = graph[1].reshape(NW, GNCH, GCHUNK)

    scatter_call, gather_call = _sc_calls()
    partials = scatter_call(edges, sdstc)

    new_nodes, p_tab, q_tab = pl.pallas_call(
        _node_body,
        grid=(N // NODE_BLK,),
        in_specs=[
            pl.BlockSpec((NODE_BLK, D), lambda i: (i, 0)),
            pl.BlockSpec((NC, NODE_BLK, D), lambda i: (0, i, 0)),
            _full((2 * D, D)),
            _full((1, D)),
            _full((D, D)),
            _full((1, D)),
            _full((D, D)),
            _full((D, D)),
        ],
        out_specs=[
            pl.BlockSpec((NODE_BLK, D), lambda i: (i, 0)),
            pl.BlockSpec((NODE_BLK, D), lambda i: (i, 0)),
            pl.BlockSpec((NODE_BLK, D), lambda i: (i, 0)),
        ],
        out_shape=[
            jax.ShapeDtypeStruct((N, D), jnp.float32),
            jax.ShapeDtypeStruct((N, D), jnp.float32),
            jax.ShapeDtypeStruct((N, D), jnp.float32),
        ],
    )(nodes, partials, node_W0, node_b0.reshape(1, D), node_W1,
      node_b1.reshape(1, D), edge_W0[:D, :], edge_W0[D:2 * D, :])

    g_sum = gather_call(p_tab, q_tab, gsrcc, gdstc)

    new_edges = pl.pallas_call(
        _edge_body,
        grid=(E // EDGE_BLK,),
        in_specs=[
            pl.BlockSpec((EDGE_BLK, D), lambda i: (i, 0)),
            pl.BlockSpec((EDGE_BLK, D), lambda i: (i, 0)),
            _full((D, D)),
            _full((1, D)),
            _full((D, D)),
            _full((1, D)),
        ],
        out_specs=pl.BlockSpec((EDGE_BLK, D), lambda i: (i, 0)),
        out_shape=jax.ShapeDtypeStruct((E, D), jnp.float32),
    )(g_sum, edges, edge_W0[2 * D:, :], edge_b0.reshape(1, D), edge_W1,
      edge_b1.reshape(1, D))

    return (new_nodes, new_edges)


# speedup vs baseline: 4.9155x; 1.2424x over previous
import functools
import jax
import jax.numpy as jnp
from jax import lax
from jax.experimental import pallas as pl
from jax.experimental.pallas import tpu as pltpu
from jax.experimental.pallas import tpu_sc as plsc

N = 10000
E = 320000
D = 128

NC = 2
NS = 16
NW = NC * NS
EW = E // NW

SCHUNK = 40
SNCH = EW // SCHUNK
NACC = N // SCHUNK
GCHUNK = 40
GNCH = EW // GCHUNK
NB = 5
NBS = 5


def _scatter_body(edges_hbm, dstc_hbm, out_hbm, idx_v, ebuf, acc,
                  sem_idx, sem_in, sem_sc):
    c = lax.axis_index("c")
    s = lax.axis_index("s")
    wid = c * NS + s
    ebase = wid * EW

    @pl.loop(0, SCHUNK)
    def _zrow(i):
        @pl.loop(0, D // 16, unroll=8)
        def _zlane(l):
            ebuf[0, i, pl.ds(l * 16, 16)] = jnp.zeros((16,), jnp.float32)

    @pl.loop(s, NACC, step=NS)
    def _zacc(k):
        pltpu.sync_copy(ebuf.at[0], acc.at[pl.ds(k * SCHUNK, SCHUNK)])

    plsc.subcore_barrier()

    for b in range(NBS):
        pltpu.async_copy(dstc_hbm.at[pl.ds((wid * SNCH + b) * SCHUNK, SCHUNK)],
                         idx_v.at[b], sem_idx.at[b])
        pltpu.async_copy(edges_hbm.at[pl.ds(ebase + b * SCHUNK, SCHUNK)],
                         ebuf.at[b], sem_in.at[b])

    @pl.loop(0, SNCH // NBS)
    def _group(g):
        for b in range(NBS):
            k = g * NBS + b
            bp = (b - 1) % NBS
            pltpu.make_async_copy(
                dstc_hbm.at[pl.ds((wid * SNCH + k) * SCHUNK, SCHUNK)],
                idx_v.at[b], sem_idx.at[b]).wait()
            pltpu.make_async_copy(
                edges_hbm.at[pl.ds(ebase + k * SCHUNK, SCHUNK)],
                ebuf.at[b], sem_in.at[b]).wait()
            pltpu.async_copy(ebuf.at[b], acc.at[idx_v.at[b]], sem_sc.at[b],
                             add=True)

            @pl.when(k >= 1)
            def _drain_prev():
                pltpu.make_async_copy(
                    ebuf.at[bp], acc.at[idx_v.at[bp]], sem_sc.at[bp]).wait()

                @pl.when(k + NBS - 1 < SNCH)
                def _refill():
                    kk = k + NBS - 1
                    pltpu.async_copy(
                        dstc_hbm.at[pl.ds((wid * SNCH + kk) * SCHUNK, SCHUNK)],
                        idx_v.at[bp], sem_idx.at[bp])
                    pltpu.async_copy(
                        edges_hbm.at[pl.ds(ebase + kk * SCHUNK, SCHUNK)],
                        ebuf.at[bp], sem_in.at[bp])

    last = (SNCH - 1) % NBS
    pltpu.make_async_copy(ebuf.at[last], acc.at[idx_v.at[last]],
                          sem_sc.at[last]).wait()

    plsc.subcore_barrier()

    @pl.loop(s, NACC, step=NS)
    def _wacc(k):
        pltpu.sync_copy(acc.at[pl.ds(k * SCHUNK, SCHUNK)],
                        out_hbm.at[c, pl.ds(k * SCHUNK, SCHUNK)])


def _gather_body(p_hbm, q_hbm, srcc_hbm, dstc_hbm, g_hbm,
                 isrc, idst, pbuf, qbuf, sem_p, sem_q, sem_out):
    c = lax.axis_index("c")
    s = lax.axis_index("s")
    wid = c * NS + s
    ebase = wid * EW

    pltpu.sync_copy(srcc_hbm.at[wid], isrc)
    pltpu.sync_copy(dstc_hbm.at[wid], idst)

    for b in range(NB):
        pltpu.async_copy(p_hbm.at[isrc.at[b]], pbuf.at[b], sem_p.at[b])
        pltpu.async_copy(q_hbm.at[idst.at[b]], qbuf.at[b], sem_q.at[b])

    @pl.loop(0, GNCH // NB)
    def _group(g):
        for b in range(NB):
            k = g * NB + b
            bp = (b - 1) % NB
            pltpu.make_async_copy(p_hbm.at[isrc.at[k]], pbuf.at[b],
                                  sem_p.at[b]).wait()
            pltpu.make_async_copy(q_hbm.at[idst.at[k]], qbuf.at[b],
                                  sem_q.at[b]).wait()

            @pl.loop(0, GCHUNK)
            def _row(i):
                @pl.loop(0, D // 16, unroll=8)
                def _lane(l):
                    sl = pl.ds(l * 16, 16)
                    pbuf[b, i, sl] = pbuf[b, i, sl] + qbuf[b, i, sl]

            pltpu.async_copy(pbuf.at[b],
                             g_hbm.at[pl.ds(ebase + k * GCHUNK, GCHUNK)],
                             sem_out.at[b])

            @pl.when(k >= 1)
            def _drain_prev():
                pltpu.make_async_copy(
                    pbuf.at[bp],
                    g_hbm.at[pl.ds(ebase + (k - 1) * GCHUNK, GCHUNK)],
                    sem_out.at[bp]).wait()

                @pl.when(k + NB - 1 < GNCH)
                def _refill():
                    kk = k + NB - 1
                    pltpu.async_copy(p_hbm.at[isrc.at[kk]], pbuf.at[bp],
                                     sem_p.at[bp])
                    pltpu.async_copy(q_hbm.at[idst.at[kk]], qbuf.at[bp],
                                     sem_q.at[bp])

    last = GNCH - 1
    pltpu.make_async_copy(pbuf.at[last % NB],
                          g_hbm.at[pl.ds(ebase + last * GCHUNK, GCHUNK)],
                          sem_out.at[last % NB]).wait()


@functools.cache
def _sc_calls():
    mesh = plsc.VectorSubcoreMesh(
        core_axis_name="c", subcore_axis_name="s",
        num_cores=NC, num_subcores=NS)
    scatter_call = pl.kernel(
        _scatter_body,
        out_type=jax.ShapeDtypeStruct((NC, N, D), jnp.float32),
        mesh=mesh,
        scratch_types=[
            pltpu.VMEM((NBS, SCHUNK), jnp.int32),
            pltpu.VMEM((NBS, SCHUNK, D), jnp.float32),
            pltpu.VMEM_SHARED((N, D), jnp.float32),
            pltpu.SemaphoreType.DMA((NBS,)),
            pltpu.SemaphoreType.DMA((NBS,)),
            pltpu.SemaphoreType.DMA((NBS,)),
        ],
    )
    gather_call = pl.kernel(
        _gather_body,
        out_type=jax.ShapeDtypeStruct((E, D), jnp.float32),
        mesh=mesh,
        scratch_types=[
            pltpu.VMEM((GNCH, GCHUNK), jnp.int32),
            pltpu.VMEM((GNCH, GCHUNK), jnp.int32),
            pltpu.VMEM((NB, GCHUNK, D), jnp.float32),
            pltpu.VMEM((NB, GCHUNK, D), jnp.float32),
            pltpu.SemaphoreType.DMA((NB,)),
            pltpu.SemaphoreType.DMA((NB,)),
            pltpu.SemaphoreType.DMA((NB,)),
        ],
    )
    return scatter_call, gather_call


NODE_BLK = 2000
EDGE_BLK = 2000


def _node_body(nodes_ref, pr_ref, w0_ref, b0_ref, w1_ref, b1_ref,
               ew0a_ref, ew0b_ref, nn_ref, p_ref, q_ref):
    msg = pr_ref[0] + pr_ref[1]
    x = nodes_ref[...]
    h = jnp.maximum(
        jnp.dot(x, w0_ref[:D, :], preferred_element_type=jnp.float32)
        + jnp.dot(msg, w0_ref[D:, :], preferred_element_type=jnp.float32)
        + b0_ref[...], 0.0)
    nn = jnp.maximum(
        jnp.dot(h, w1_ref[...], preferred_element_type=jnp.float32)
        + b1_ref[...], 0.0) + x
    nn_ref[...] = nn
    p_ref[...] = jnp.dot(nn, ew0a_ref[...], preferred_element_type=jnp.float32)
    q_ref[...] = jnp.dot(nn, ew0b_ref[...], preferred_element_type=jnp.float32)


def _edge_body(g_ref, edges_ref, ew0c_ref, eb0_ref, ew1_ref, eb1_ref,
               out_ref):
    e = edges_ref[...]
    he = jnp.maximum(
        g_ref[...]
        + jnp.dot(e, ew0c_ref[...], preferred_element_type=jnp.float32)
        + eb0_ref[...], 0.0)
    out_ref[...] = jnp.tanh(
        jnp.dot(he, ew1_ref[...], preferred_element_type=jnp.float32)
        + eb1_ref[...]) + e


def _full(shape):
    return pl.BlockSpec(shape, lambda i: (0,) * len(shape))


def kernel(nodes, edges, graph, node_W0, node_b0, node_W1, node_b1,
           edge_W0, edge_b0, edge_W1, edge_b1):
    dst_flat = graph[1]
    gsrcc = graph[0].reshape(NW, GNCH, GCHUNK)
    gdstc = graph[1].reshape(NW, GNCH, GCHUNK)

    scatter_call, gather_call = _sc_calls()
    partials = scatter_call(edges, dst_flat)

    new_nodes, p_tab, q_tab = pl.pallas_call(
        _node_body,
        grid=(N // NODE_BLK,),
        in_specs=[
            pl.BlockSpec((NODE_BLK, D), lambda i: (i, 0)),
            pl.BlockSpec((NC, NODE_BLK, D), lambda i: (0, i, 0)),
            _full((2 * D, D)),
            _full((1, D)),
            _full((D, D)),
            _full((1, D)),
            _full((D, D)),
            _full((D, D)),
        ],
        out_specs=[
            pl.BlockSpec((NODE_BLK, D), lambda i: (i, 0)),
            pl.BlockSpec((NODE_BLK, D), lambda i: (i, 0)),
            pl.BlockSpec((NODE_BLK, D), lambda i: (i, 0)),
        ],
        out_shape=[
            jax.ShapeDtypeStruct((N, D), jnp.float32),
            jax.ShapeDtypeStruct((N, D), jnp.float32),
            jax.ShapeDtypeStruct((N, D), jnp.float32),
        ],
    )(nodes, partials, node_W0, node_b0.reshape(1, D), node_W1,
      node_b1.reshape(1, D), edge_W0[:D, :], edge_W0[D:2 * D, :])

    g_sum = gather_call(p_tab, q_tab, gsrcc, gdstc)

    new_edges = pl.pallas_call(
        _edge_body,
        grid=(E // EDGE_BLK,),
        in_specs=[
            pl.BlockSpec((EDGE_BLK, D), lambda i: (i, 0)),
            pl.BlockSpec((EDGE_BLK, D), lambda i: (i, 0)),
            _full((D, D)),
            _full((1, D)),
            _full((D, D)),
            _full((1, D)),
        ],
        out_specs=pl.BlockSpec((EDGE_BLK, D), lambda i: (i, 0)),
        out_shape=jax.ShapeDtypeStruct((E, D), jnp.float32),
    )(g_sum, edges, edge_W0[2 * D:, :], edge_b0.reshape(1, D), edge_W1,
      edge_b1.reshape(1, D))

    return (new_nodes, new_edges)


# speedup vs baseline: 5.1324x; 1.0441x over previous
import functools
import jax
import jax.numpy as jnp
from jax import lax
from jax.experimental import pallas as pl
from jax.experimental.pallas import tpu as pltpu
from jax.experimental.pallas import tpu_sc as plsc

N = 10000
E = 320000
D = 128

NC = 2
NS = 16
NW = NC * NS
EW = E // NW

SCHUNK = 40
SNCH = EW // SCHUNK
NACC = N // SCHUNK
GCHUNK = 40
NSLICE = 2
ES = E // NSLICE
EWG = ES // NW
GNCH = EWG // GCHUNK
NB = 5
NBS = 5


def _scatter_body(edges_hbm, dstc_hbm, out_hbm, idx_v, ebuf, acc,
                  sem_idx, sem_in, sem_sc):
    c = lax.axis_index("c")
    s = lax.axis_index("s")
    wid = c * NS + s
    ebase = wid * EW

    @pl.loop(0, SCHUNK)
    def _zrow(i):
        @pl.loop(0, D // 16, unroll=8)
        def _zlane(l):
            ebuf[0, i, pl.ds(l * 16, 16)] = jnp.zeros((16,), jnp.float32)

    @pl.loop(s, NACC, step=NS)
    def _zacc(k):
        pltpu.sync_copy(ebuf.at[0], acc.at[pl.ds(k * SCHUNK, SCHUNK)])

    plsc.subcore_barrier()

    for b in range(NBS):
        pltpu.async_copy(dstc_hbm.at[pl.ds((wid * SNCH + b) * SCHUNK, SCHUNK)],
                         idx_v.at[b], sem_idx.at[b])
        pltpu.async_copy(edges_hbm.at[pl.ds(ebase + b * SCHUNK, SCHUNK)],
                         ebuf.at[b], sem_in.at[b])

    @pl.loop(0, SNCH // NBS)
    def _group(g):
        for b in range(NBS):
            k = g * NBS + b
            bp = (b - 1) % NBS
            pltpu.make_async_copy(
                dstc_hbm.at[pl.ds((wid * SNCH + k) * SCHUNK, SCHUNK)],
                idx_v.at[b], sem_idx.at[b]).wait()
            pltpu.make_async_copy(
                edges_hbm.at[pl.ds(ebase + k * SCHUNK, SCHUNK)],
                ebuf.at[b], sem_in.at[b]).wait()
            pltpu.async_copy(ebuf.at[b], acc.at[idx_v.at[b]], sem_sc.at[b],
                             add=True)

            @pl.when(k >= 1)
            def _drain_prev():
                pltpu.make_async_copy(
                    ebuf.at[bp], acc.at[idx_v.at[bp]], sem_sc.at[bp]).wait()

                @pl.when(k + NBS - 1 < SNCH)
                def _refill():
                    kk = k + NBS - 1
                    pltpu.async_copy(
                        dstc_hbm.at[pl.ds((wid * SNCH + kk) * SCHUNK, SCHUNK)],
                        idx_v.at[bp], sem_idx.at[bp])
                    pltpu.async_copy(
                        edges_hbm.at[pl.ds(ebase + kk * SCHUNK, SCHUNK)],
                        ebuf.at[bp], sem_in.at[bp])

    last = (SNCH - 1) % NBS
    pltpu.make_async_copy(ebuf.at[last], acc.at[idx_v.at[last]],
                          sem_sc.at[last]).wait()

    plsc.subcore_barrier()

    @pl.loop(s, NACC, step=NS)
    def _wacc(k):
        pltpu.sync_copy(acc.at[pl.ds(k * SCHUNK, SCHUNK)],
                        out_hbm.at[c, pl.ds(k * SCHUNK, SCHUNK)])


def _gather_body(p_hbm, q_hbm, srcc_hbm, dstc_hbm, g_hbm,
                 isrc, idst, pbuf, qbuf, sem_p, sem_q, sem_out):
    c = lax.axis_index("c")
    s = lax.axis_index("s")
    wid = c * NS + s
    ebase = wid * EWG

    pltpu.sync_copy(srcc_hbm.at[wid], isrc)
    pltpu.sync_copy(dstc_hbm.at[wid], idst)

    for b in range(NB):
        pltpu.async_copy(p_hbm.at[isrc.at[b]], pbuf.at[b], sem_p.at[b])
        pltpu.async_copy(q_hbm.at[idst.at[b]], qbuf.at[b], sem_q.at[b])

    @pl.loop(0, GNCH // NB)
    def _group(g):
        for b in range(NB):
            k = g * NB + b
            bp = (b - 1) % NB
            pltpu.make_async_copy(p_hbm.at[isrc.at[k]], pbuf.at[b],
                                  sem_p.at[b]).wait()
            pltpu.make_async_copy(q_hbm.at[idst.at[k]], qbuf.at[b],
                                  sem_q.at[b]).wait()

            @pl.loop(0, GCHUNK)
            def _row(i):
                @pl.loop(0, D // 16, unroll=8)
                def _lane(l):
                    sl = pl.ds(l * 16, 16)
                    pbuf[b, i, sl] = pbuf[b, i, sl] + qbuf[b, i, sl]

            pltpu.async_copy(pbuf.at[b],
                             g_hbm.at[pl.ds(ebase + k * GCHUNK, GCHUNK)],
                             sem_out.at[b])

            @pl.when(k >= 1)
            def _drain_prev():
                pltpu.make_async_copy(
                    pbuf.at[bp],
                    g_hbm.at[pl.ds(ebase + (k - 1) * GCHUNK, GCHUNK)],
                    sem_out.at[bp]).wait()

                @pl.when(k + NB - 1 < GNCH)
                def _refill():
                    kk = k + NB - 1
                    pltpu.async_copy(p_hbm.at[isrc.at[kk]], pbuf.at[bp],
                                     sem_p.at[bp])
                    pltpu.async_copy(q_hbm.at[idst.at[kk]], qbuf.at[bp],
                                     sem_q.at[bp])

    last = GNCH - 1
    pltpu.make_async_copy(pbuf.at[last % NB],
                          g_hbm.at[pl.ds(ebase + last * GCHUNK, GCHUNK)],
                          sem_out.at[last % NB]).wait()


@functools.cache
def _sc_calls():
    mesh = plsc.VectorSubcoreMesh(
        core_axis_name="c", subcore_axis_name="s",
        num_cores=NC, num_subcores=NS)
    scatter_call = pl.kernel(
        _scatter_body,
        out_type=jax.ShapeDtypeStruct((NC, N, D), jnp.float32),
        mesh=mesh,
        scratch_types=[
            pltpu.VMEM((NBS, SCHUNK), jnp.int32),
            pltpu.VMEM((NBS, SCHUNK, D), jnp.float32),
            pltpu.VMEM_SHARED((N, D), jnp.float32),
            pltpu.SemaphoreType.DMA((NBS,)),
            pltpu.SemaphoreType.DMA((NBS,)),
            pltpu.SemaphoreType.DMA((NBS,)),
        ],
    )
    gather_call = pl.kernel(
        _gather_body,
        out_type=jax.ShapeDtypeStruct((ES, D), jnp.float32),
        mesh=mesh,
        scratch_types=[
            pltpu.VMEM((GNCH, GCHUNK), jnp.int32),
            pltpu.VMEM((GNCH, GCHUNK), jnp.int32),
            pltpu.VMEM((NB, GCHUNK, D), jnp.float32),
            pltpu.VMEM((NB, GCHUNK, D), jnp.float32),
            pltpu.SemaphoreType.DMA((NB,)),
            pltpu.SemaphoreType.DMA((NB,)),
            pltpu.SemaphoreType.DMA((NB,)),
        ],
    )
    return scatter_call, gather_call


NODE_BLK = 2000
EDGE_BLK = 2000


def _node_body(nodes_ref, pr_ref, w0_ref, b0_ref, w1_ref, b1_ref,
               ew0a_ref, ew0b_ref, nn_ref, p_ref, q_ref):
    msg = pr_ref[0] + pr_ref[1]
    x = nodes_ref[...]
    h = jnp.maximum(
        jnp.dot(x, w0_ref[:D, :], preferred_element_type=jnp.float32)
        + jnp.dot(msg, w0_ref[D:, :], preferred_element_type=jnp.float32)
        + b0_ref[...], 0.0)
    nn = jnp.maximum(
        jnp.dot(h, w1_ref[...], preferred_element_type=jnp.float32)
        + b1_ref[...], 0.0) + x
    nn_ref[...] = nn
    p_ref[...] = jnp.dot(nn, ew0a_ref[...], preferred_element_type=jnp.float32)
    q_ref[...] = jnp.dot(nn, ew0b_ref[...], preferred_element_type=jnp.float32)


def _edge_body(g_ref, edges_ref, ew0c_ref, eb0_ref, ew1_ref, eb1_ref,
               out_ref):
    e = edges_ref[...]
    he = jnp.maximum(
        g_ref[...]
        + jnp.dot(e, ew0c_ref[...], preferred_element_type=jnp.float32)
        + eb0_ref[...], 0.0)
    out_ref[...] = jnp.tanh(
        jnp.dot(he, ew1_ref[...], preferred_element_type=jnp.float32)
        + eb1_ref[...]) + e


def _edge_body_alias(g_ref, edges_ref, ew0c_ref, eb0_ref, ew1_ref, eb1_ref,
                     prev_ref, out_ref):
    del prev_ref
    _edge_body(g_ref, edges_ref, ew0c_ref, eb0_ref, ew1_ref, eb1_ref, out_ref)


def _full(shape):
    return pl.BlockSpec(shape, lambda i: (0,) * len(shape))


def kernel(nodes, edges, graph, node_W0, node_b0, node_W1, node_b1,
           edge_W0, edge_b0, edge_W1, edge_b1):
    dst_flat = graph[1]
    gsrcc = graph[0].reshape(NSLICE, NW, GNCH, GCHUNK)
    gdstc = graph[1].reshape(NSLICE, NW, GNCH, GCHUNK)

    scatter_call, gather_call = _sc_calls()
    partials = scatter_call(edges, dst_flat)

    new_nodes, p_tab, q_tab = pl.pallas_call(
        _node_body,
        grid=(N // NODE_BLK,),
        in_specs=[
            pl.BlockSpec((NODE_BLK, D), lambda i: (i, 0)),
            pl.BlockSpec((NC, NODE_BLK, D), lambda i: (0, i, 0)),
            _full((2 * D, D)),
            _full((1, D)),
            _full((D, D)),
            _full((1, D)),
            _full((D, D)),
            _full((D, D)),
        ],
        out_specs=[
            pl.BlockSpec((NODE_BLK, D), lambda i: (i, 0)),
            pl.BlockSpec((NODE_BLK, D), lambda i: (i, 0)),
            pl.BlockSpec((NODE_BLK, D), lambda i: (i, 0)),
        ],
        out_shape=[
            jax.ShapeDtypeStruct((N, D), jnp.float32),
            jax.ShapeDtypeStruct((N, D), jnp.float32),
            jax.ShapeDtypeStruct((N, D), jnp.float32),
        ],
    )(nodes, partials, node_W0, node_b0.reshape(1, D), node_W1,
      node_b1.reshape(1, D), edge_W0[:D, :], edge_W0[D:2 * D, :])

    ew0c = edge_W0[2 * D:, :]
    eb0 = edge_b0.reshape(1, D)
    eb1 = edge_b1.reshape(1, D)
    blk_per_slice = ES // EDGE_BLK

    g_slices = [gather_call(p_tab, q_tab, gsrcc[i], gdstc[i])
                for i in range(NSLICE)]

    new_edges = None
    for i in range(NSLICE):
        base = i * blk_per_slice
        edge_specs = [
            pl.BlockSpec((EDGE_BLK, D), lambda j: (j, 0)),
            pl.BlockSpec((EDGE_BLK, D),
                         functools.partial(lambda b, j: (b + j, 0), base)),
            _full((D, D)),
            _full((1, D)),
            _full((D, D)),
            _full((1, D)),
        ]
        out_spec = pl.BlockSpec((EDGE_BLK, D),
                                functools.partial(lambda b, j: (b + j, 0),
                                                  base))
        if i == 0:
            new_edges = pl.pallas_call(
                _edge_body,
                grid=(blk_per_slice,),
                in_specs=edge_specs,
                out_specs=out_spec,
                out_shape=jax.ShapeDtypeStruct((E, D), jnp.float32),
            )(g_slices[i], edges, ew0c, eb0, edge_W1, eb1)
        else:
            new_edges = pl.pallas_call(
                _edge_body_alias,
                grid=(blk_per_slice,),
                in_specs=edge_specs + [pl.BlockSpec((8, D), lambda j: (0, 0))],
                out_specs=out_spec,
                out_shape=jax.ShapeDtypeStruct((E, D), jnp.float32),
                input_output_aliases={6: 0},
            )(g_slices[i], edges, ew0c, eb0, edge_W1, eb1, new_edges)

    return (new_nodes, new_edges)


# speedup vs baseline: 5.2192x; 1.0169x over previous
import functools
import jax
import jax.numpy as jnp
from jax import lax
from jax.experimental import pallas as pl
from jax.experimental.pallas import tpu as pltpu
from jax.experimental.pallas import tpu_sc as plsc

N = 10000
E = 320000
D = 128

NC = 2
NS = 16
NW = NC * NS
EW = E // NW

SCHUNK = 40
SNCH = EW // SCHUNK
NACC = N // SCHUNK
GCHUNK = 40
NSLICE = 5
ES = E // NSLICE
EWG = ES // NW
GNCH = EWG // GCHUNK
NB = 5
NBS = 5


def _scatter_body(edges_hbm, dstc_hbm, out_hbm, idx_v, ebuf, acc,
                  sem_idx, sem_in, sem_sc):
    c = lax.axis_index("c")
    s = lax.axis_index("s")
    wid = c * NS + s
    ebase = wid * EW

    @pl.loop(0, SCHUNK)
    def _zrow(i):
        @pl.loop(0, D // 16, unroll=8)
        def _zlane(l):
            ebuf[0, i, pl.ds(l * 16, 16)] = jnp.zeros((16,), jnp.float32)

    @pl.loop(s, NACC, step=NS)
    def _zacc(k):
        pltpu.sync_copy(ebuf.at[0], acc.at[pl.ds(k * SCHUNK, SCHUNK)])

    plsc.subcore_barrier()

    for b in range(NBS):
        pltpu.async_copy(dstc_hbm.at[pl.ds((wid * SNCH + b) * SCHUNK, SCHUNK)],
                         idx_v.at[b], sem_idx.at[b])
        pltpu.async_copy(edges_hbm.at[pl.ds(ebase + b * SCHUNK, SCHUNK)],
                         ebuf.at[b], sem_in.at[b])

    @pl.loop(0, SNCH // NBS)
    def _group(g):
        for b in range(NBS):
            k = g * NBS + b
            bp = (b - 1) % NBS
            pltpu.make_async_copy(
                dstc_hbm.at[pl.ds((wid * SNCH + k) * SCHUNK, SCHUNK)],
                idx_v.at[b], sem_idx.at[b]).wait()
            pltpu.make_async_copy(
                edges_hbm.at[pl.ds(ebase + k * SCHUNK, SCHUNK)],
                ebuf.at[b], sem_in.at[b]).wait()
            pltpu.async_copy(ebuf.at[b], acc.at[idx_v.at[b]], sem_sc.at[b],
                             add=True)

            @pl.when(k >= 1)
            def _drain_prev():
                pltpu.make_async_copy(
                    ebuf.at[bp], acc.at[idx_v.at[bp]], sem_sc.at[bp]).wait()

                @pl.when(k + NBS - 1 < SNCH)
                def _refill():
                    kk = k + NBS - 1
                    pltpu.async_copy(
                        dstc_hbm.at[pl.ds((wid * SNCH + kk) * SCHUNK, SCHUNK)],
                        idx_v.at[bp], sem_idx.at[bp])
                    pltpu.async_copy(
                        edges_hbm.at[pl.ds(ebase + kk * SCHUNK, SCHUNK)],
                        ebuf.at[bp], sem_in.at[bp])

    last = (SNCH - 1) % NBS
    pltpu.make_async_copy(ebuf.at[last], acc.at[idx_v.at[last]],
                          sem_sc.at[last]).wait()

    plsc.subcore_barrier()

    @pl.loop(s, NACC, step=NS)
    def _wacc(k):
        pltpu.sync_copy(acc.at[pl.ds(k * SCHUNK, SCHUNK)],
                        out_hbm.at[c, pl.ds(k * SCHUNK, SCHUNK)])


def _gather_body(p_hbm, q_hbm, srcc_hbm, dstc_hbm, g_hbm,
                 isrc, idst, pbuf, qbuf, sem_p, sem_q, sem_out):
    c = lax.axis_index("c")
    s = lax.axis_index("s")
    wid = c * NS + s
    ebase = wid * EWG

    pltpu.sync_copy(srcc_hbm.at[wid], isrc)
    pltpu.sync_copy(dstc_hbm.at[wid], idst)

    for b in range(NB):
        pltpu.async_copy(p_hbm.at[isrc.at[b]], pbuf.at[b], sem_p.at[b])
        pltpu.async_copy(q_hbm.at[idst.at[b]], qbuf.at[b], sem_q.at[b])

    @pl.loop(0, GNCH // NB)
    def _group(g):
        for b in range(NB):
            k = g * NB + b
            bp = (b - 1) % NB
            pltpu.make_async_copy(p_hbm.at[isrc.at[k]], pbuf.at[b],
                                  sem_p.at[b]).wait()
            pltpu.make_async_copy(q_hbm.at[idst.at[k]], qbuf.at[b],
                                  sem_q.at[b]).wait()

            @pl.loop(0, GCHUNK)
            def _row(i):
                @pl.loop(0, D // 16, unroll=8)
                def _lane(l):
                    sl = pl.ds(l * 16, 16)
                    pbuf[b, i, sl] = pbuf[b, i, sl] + qbuf[b, i, sl]

            pltpu.async_copy(pbuf.at[b],
                             g_hbm.at[pl.ds(ebase + k * GCHUNK, GCHUNK)],
                             sem_out.at[b])

            @pl.when(k >= 1)
            def _drain_prev():
                pltpu.make_async_copy(
                    pbuf.at[bp],
                    g_hbm.at[pl.ds(ebase + (k - 1) * GCHUNK, GCHUNK)],
                    sem_out.at[bp]).wait()

                @pl.when(k + NB - 1 < GNCH)
                def _refill():
                    kk = k + NB - 1
                    pltpu.async_copy(p_hbm.at[isrc.at[kk]], pbuf.at[bp],
                                     sem_p.at[bp])
                    pltpu.async_copy(q_hbm.at[idst.at[kk]], qbuf.at[bp],
                                     sem_q.at[bp])

    last = GNCH - 1
    pltpu.make_async_copy(pbuf.at[last % NB],
                          g_hbm.at[pl.ds(ebase + last * GCHUNK, GCHUNK)],
                          sem_out.at[last % NB]).wait()


@functools.cache
def _sc_calls():
    mesh = plsc.VectorSubcoreMesh(
        core_axis_name="c", subcore_axis_name="s",
        num_cores=NC, num_subcores=NS)
    scatter_call = pl.kernel(
        _scatter_body,
        out_type=jax.ShapeDtypeStruct((NC, N, D), jnp.float32),
        mesh=mesh,
        scratch_types=[
            pltpu.VMEM((NBS, SCHUNK), jnp.int32),
            pltpu.VMEM((NBS, SCHUNK, D), jnp.float32),
            pltpu.VMEM_SHARED((N, D), jnp.float32),
            pltpu.SemaphoreType.DMA((NBS,)),
            pltpu.SemaphoreType.DMA((NBS,)),
            pltpu.SemaphoreType.DMA((NBS,)),
        ],
    )
    gather_call = pl.kernel(
        _gather_body,
        out_type=jax.ShapeDtypeStruct((ES, D), jnp.float32),
        mesh=mesh,
        scratch_types=[
            pltpu.VMEM((GNCH, GCHUNK), jnp.int32),
            pltpu.VMEM((GNCH, GCHUNK), jnp.int32),
            pltpu.VMEM((NB, GCHUNK, D), jnp.float32),
            pltpu.VMEM((NB, GCHUNK, D), jnp.float32),
            pltpu.SemaphoreType.DMA((NB,)),
            pltpu.SemaphoreType.DMA((NB,)),
            pltpu.SemaphoreType.DMA((NB,)),
        ],
    )
    return scatter_call, gather_call


NODE_BLK = 2000
EDGE_BLK = 2000


def _node_body(nodes_ref, pr_ref, w0_ref, b0_ref, w1_ref, b1_ref,
               ew0a_ref, ew0b_ref, nn_ref, p_ref, q_ref):
    msg = pr_ref[0] + pr_ref[1]
    x = nodes_ref[...]
    h = jnp.maximum(
        jnp.dot(x, w0_ref[:D, :], preferred_element_type=jnp.float32)
        + jnp.dot(msg, w0_ref[D:, :], preferred_element_type=jnp.float32)
        + b0_ref[...], 0.0)
    nn = jnp.maximum(
        jnp.dot(h, w1_ref[...], preferred_element_type=jnp.float32)
        + b1_ref[...], 0.0) + x
    nn_ref[...] = nn
    p_ref[...] = jnp.dot(nn, ew0a_ref[...], preferred_element_type=jnp.float32)
    q_ref[...] = jnp.dot(nn, ew0b_ref[...], preferred_element_type=jnp.float32)


def _edge_body(g_ref, edges_ref, ew0c_ref, eb0_ref, ew1_ref, eb1_ref,
               out_ref):
    e = edges_ref[...]
    he = jnp.maximum(
        g_ref[...]
        + jnp.dot(e, ew0c_ref[...], preferred_element_type=jnp.float32)
        + eb0_ref[...], 0.0)
    out_ref[...] = jnp.tanh(
        jnp.dot(he, ew1_ref[...], preferred_element_type=jnp.float32)
        + eb1_ref[...]) + e


def _edge_body_alias(g_ref, edges_ref, ew0c_ref, eb0_ref, ew1_ref, eb1_ref,
                     prev_ref, out_ref):
    del prev_ref
    _edge_body(g_ref, edges_ref, ew0c_ref, eb0_ref, ew1_ref, eb1_ref, out_ref)


def _full(shape):
    return pl.BlockSpec(shape, lambda i: (0,) * len(shape))


def kernel(nodes, edges, graph, node_W0, node_b0, node_W1, node_b1,
           edge_W0, edge_b0, edge_W1, edge_b1):
    dst_flat = graph[1]
    gsrcc = graph[0].reshape(NSLICE, NW, GNCH, GCHUNK)
    gdstc = graph[1].reshape(NSLICE, NW, GNCH, GCHUNK)

    scatter_call, gather_call = _sc_calls()
    partials = scatter_call(edges, dst_flat)

    new_nodes, p_tab, q_tab = pl.pallas_call(
        _node_body,
        grid=(N // NODE_BLK,),
        in_specs=[
            pl.BlockSpec((NODE_BLK, D), lambda i: (i, 0)),
            pl.BlockSpec((NC, NODE_BLK, D), lambda i: (0, i, 0)),
            _full((2 * D, D)),
            _full((1, D)),
            _full((D, D)),
            _full((1, D)),
            _full((D, D)),
            _full((D, D)),
        ],
        out_specs=[
            pl.BlockSpec((NODE_BLK, D), lambda i: (i, 0)),
            pl.BlockSpec((NODE_BLK, D), lambda i: (i, 0)),
            pl.BlockSpec((NODE_BLK, D), lambda i: (i, 0)),
        ],
        out_shape=[
            jax.ShapeDtypeStruct((N, D), jnp.float32),
            jax.ShapeDtypeStruct((N, D), jnp.float32),
            jax.ShapeDtypeStruct((N, D), jnp.float32),
        ],
    )(nodes, partials, node_W0, node_b0.reshape(1, D), node_W1,
      node_b1.reshape(1, D), edge_W0[:D, :], edge_W0[D:2 * D, :])

    ew0c = edge_W0[2 * D:, :]
    eb0 = edge_b0.reshape(1, D)
    eb1 = edge_b1.reshape(1, D)
    blk_per_slice = ES // EDGE_BLK

    g_slices = [gather_call(p_tab, q_tab, gsrcc[i], gdstc[i])
                for i in range(NSLICE)]

    new_edges = None
    for i in range(NSLICE):
        base = i * blk_per_slice
        edge_specs = [
            pl.BlockSpec((EDGE_BLK, D), lambda j: (j, 0)),
            pl.BlockSpec((EDGE_BLK, D),
                         functools.partial(lambda b, j: (b + j, 0), base)),
            _full((D, D)),
            _full((1, D)),
            _full((D, D)),
            _full((1, D)),
        ]
        out_spec = pl.BlockSpec((EDGE_BLK, D),
                                functools.partial(lambda b, j: (b + j, 0),
                                                  base))
        if i == 0:
            new_edges = pl.pallas_call(
                _edge_body,
                grid=(blk_per_slice,),
                in_specs=edge_specs,
                out_specs=out_spec,
                out_shape=jax.ShapeDtypeStruct((E, D), jnp.float32),
            )(g_slices[i], edges, ew0c, eb0, edge_W1, eb1)
        else:
            new_edges = pl.pallas_call(
                _edge_body_alias,
                grid=(blk_per_slice,),
                in_specs=edge_specs + [pl.BlockSpec((8, D), lambda j: (0, 0))],
                out_specs=out_spec,
                out_shape=jax.ShapeDtypeStruct((E, D), jnp.float32),
                input_output_aliases={6: 0},
            )(g_slices[i], edges, ew0c, eb0, edge_W1, eb1, new_edges)

    return (new_nodes, new_edges)


# speedup vs baseline: 5.3507x; 1.0252x over previous
import functools
import jax
import jax.numpy as jnp
from jax import lax
from jax.experimental import pallas as pl
from jax.experimental.pallas import tpu as pltpu
from jax.experimental.pallas import tpu_sc as plsc

N = 10000
E = 320000
D = 128

NC = 2
NS = 16
NW = NC * NS
EW = E // NW

SCHUNK = 40
SNCH = EW // SCHUNK
NACC = N // SCHUNK
GCHUNK = 40
NSLICE = 5
ES = E // NSLICE
EWG = ES // NW
GNCH = EWG // GCHUNK
NB = 5
NBS = 5


def _scatter_body(edges_hbm, dstc_hbm, out_hbm, idx_v, ebuf, acc,
                  sem_idx, sem_in, sem_sc):
    c = lax.axis_index("c")
    s = lax.axis_index("s")
    wid = c * NS + s
    ebase = wid * EW

    @pl.loop(0, SCHUNK)
    def _zrow(i):
        @pl.loop(0, D // 16, unroll=8)
        def _zlane(l):
            ebuf[0, i, pl.ds(l * 16, 16)] = jnp.zeros((16,), jnp.float32)

    @pl.loop(s, NACC, step=NS)
    def _zacc(k):
        pltpu.sync_copy(ebuf.at[0], acc.at[pl.ds(k * SCHUNK, SCHUNK)])

    plsc.subcore_barrier()

    for b in range(NBS):
        pltpu.async_copy(dstc_hbm.at[pl.ds((wid * SNCH + b) * SCHUNK, SCHUNK)],
                         idx_v.at[b], sem_idx.at[b])
        pltpu.async_copy(edges_hbm.at[pl.ds(ebase + b * SCHUNK, SCHUNK)],
                         ebuf.at[b], sem_in.at[b])

    @pl.loop(0, SNCH // NBS)
    def _group(g):
        for b in range(NBS):
            k = g * NBS + b
            bp = (b - 1) % NBS
            pltpu.make_async_copy(
                dstc_hbm.at[pl.ds((wid * SNCH + k) * SCHUNK, SCHUNK)],
                idx_v.at[b], sem_idx.at[b]).wait()
            pltpu.make_async_copy(
                edges_hbm.at[pl.ds(ebase + k * SCHUNK, SCHUNK)],
                ebuf.at[b], sem_in.at[b]).wait()
            pltpu.async_copy(ebuf.at[b], acc.at[idx_v.at[b]], sem_sc.at[b],
                             add=True)

            @pl.when(k >= 1)
            def _drain_prev():
                pltpu.make_async_copy(
                    ebuf.at[bp], acc.at[idx_v.at[bp]], sem_sc.at[bp]).wait()

                @pl.when(k + NBS - 1 < SNCH)
                def _refill():
                    kk = k + NBS - 1
                    pltpu.async_copy(
                        dstc_hbm.at[pl.ds((wid * SNCH + kk) * SCHUNK, SCHUNK)],
                        idx_v.at[bp], sem_idx.at[bp])
                    pltpu.async_copy(
                        edges_hbm.at[pl.ds(ebase + kk * SCHUNK, SCHUNK)],
                        ebuf.at[bp], sem_in.at[bp])

    last = (SNCH - 1) % NBS
    pltpu.make_async_copy(ebuf.at[last], acc.at[idx_v.at[last]],
                          sem_sc.at[last]).wait()

    plsc.subcore_barrier()

    @pl.loop(s, NACC, step=NS)
    def _wacc(k):
        pltpu.sync_copy(acc.at[pl.ds(k * SCHUNK, SCHUNK)],
                        out_hbm.at[c, pl.ds(k * SCHUNK, SCHUNK)])


def _gather_body(p_hbm, q_hbm, srcc_hbm, dstc_hbm, g_hbm,
                 isrc, idst, pbuf, qbuf, sem_p, sem_q, sem_out):
    c = lax.axis_index("c")
    s = lax.axis_index("s")
    wid = c * NS + s
    ebase = wid * EWG

    pltpu.sync_copy(srcc_hbm.at[wid], isrc)
    pltpu.sync_copy(dstc_hbm.at[wid], idst)

    for b in range(NB):
        pltpu.async_copy(p_hbm.at[isrc.at[b]], pbuf.at[b], sem_p.at[b])
        pltpu.async_copy(q_hbm.at[idst.at[b]], qbuf.at[b], sem_q.at[b])

    @pl.loop(0, GNCH // NB)
    def _group(g):
        for b in range(NB):
            k = g * NB + b
            bp = (b - 1) % NB
            pltpu.make_async_copy(p_hbm.at[isrc.at[k]], pbuf.at[b],
                                  sem_p.at[b]).wait()
            pltpu.make_async_copy(q_hbm.at[idst.at[k]], qbuf.at[b],
                                  sem_q.at[b]).wait()

            @pl.loop(0, GCHUNK)
            def _row(i):
                @pl.loop(0, D // 16, unroll=8)
                def _lane(l):
                    sl = pl.ds(l * 16, 16)
                    pbuf[b, i, sl] = pbuf[b, i, sl] + qbuf[b, i, sl]

            pltpu.async_copy(pbuf.at[b],
                             g_hbm.at[pl.ds(ebase + k * GCHUNK, GCHUNK)],
                             sem_out.at[b])

            @pl.when(k >= 1)
            def _drain_prev():
                pltpu.make_async_copy(
                    pbuf.at[bp],
                    g_hbm.at[pl.ds(ebase + (k - 1) * GCHUNK, GCHUNK)],
                    sem_out.at[bp]).wait()

                @pl.when(k + NB - 1 < GNCH)
                def _refill():
                    kk = k + NB - 1
                    pltpu.async_copy(p_hbm.at[isrc.at[kk]], pbuf.at[bp],
                                     sem_p.at[bp])
                    pltpu.async_copy(q_hbm.at[idst.at[kk]], qbuf.at[bp],
                                     sem_q.at[bp])

    last = GNCH - 1
    pltpu.make_async_copy(pbuf.at[last % NB],
                          g_hbm.at[pl.ds(ebase + last * GCHUNK, GCHUNK)],
                          sem_out.at[last % NB]).wait()


@functools.cache
def _sc_calls():
    mesh = plsc.VectorSubcoreMesh(
        core_axis_name="c", subcore_axis_name="s",
        num_cores=NC, num_subcores=NS)
    scatter_call = pl.kernel(
        _scatter_body,
        out_type=jax.ShapeDtypeStruct((NC, N, D), jnp.float32),
        mesh=mesh,
        scratch_types=[
            pltpu.VMEM((NBS, SCHUNK), jnp.int32),
            pltpu.VMEM((NBS, SCHUNK, D), jnp.float32),
            pltpu.VMEM_SHARED((N, D), jnp.float32),
            pltpu.SemaphoreType.DMA((NBS,)),
            pltpu.SemaphoreType.DMA((NBS,)),
            pltpu.SemaphoreType.DMA((NBS,)),
        ],
    )
    gather_call = pl.kernel(
        _gather_body,
        out_type=jax.ShapeDtypeStruct((ES, D), jnp.float32),
        mesh=mesh,
        scratch_types=[
            pltpu.VMEM((GNCH, GCHUNK), jnp.int32),
            pltpu.VMEM((GNCH, GCHUNK), jnp.int32),
            pltpu.VMEM((NB, GCHUNK, D), jnp.float32),
            pltpu.VMEM((NB, GCHUNK, D), jnp.float32),
            pltpu.SemaphoreType.DMA((NB,)),
            pltpu.SemaphoreType.DMA((NB,)),
            pltpu.SemaphoreType.DMA((NB,)),
        ],
    )
    return scatter_call, gather_call


NODE_BLK = 2000
EDGE_BLK = 4000


def _node_body(nodes_ref, pr_ref, w0_ref, b0_ref, w1_ref, b1_ref,
               ew0a_ref, ew0b_ref, nn_ref, p_ref, q_ref):
    msg = pr_ref[0] + pr_ref[1]
    x = nodes_ref[...]
    h = jnp.maximum(
        jnp.dot(x, w0_ref[:D, :], preferred_element_type=jnp.float32)
        + jnp.dot(msg, w0_ref[D:, :], preferred_element_type=jnp.float32)
        + b0_ref[...], 0.0)
    nn = jnp.maximum(
        jnp.dot(h, w1_ref[...], preferred_element_type=jnp.float32)
        + b1_ref[...], 0.0) + x
    nn_ref[...] = nn
    p_ref[...] = jnp.dot(nn, ew0a_ref[...], preferred_element_type=jnp.float32)
    q_ref[...] = jnp.dot(nn, ew0b_ref[...], preferred_element_type=jnp.float32)


def _edge_body(g_ref, edges_ref, ew0c_ref, eb0_ref, ew1_ref, eb1_ref,
               out_ref):
    e = edges_ref[...]
    he = jnp.maximum(
        g_ref[...]
        + jnp.dot(e, ew0c_ref[...], preferred_element_type=jnp.float32)
        + eb0_ref[...], 0.0)
    out_ref[...] = jnp.tanh(
        jnp.dot(he, ew1_ref[...], preferred_element_type=jnp.float32)
        + eb1_ref[...]) + e


def _edge_body_alias(g_ref, edges_ref, ew0c_ref, eb0_ref, ew1_ref, eb1_ref,
                     prev_ref, out_ref):
    del prev_ref
    _edge_body(g_ref, edges_ref, ew0c_ref, eb0_ref, ew1_ref, eb1_ref, out_ref)


def _full(shape):
    return pl.BlockSpec(shape, lambda i: (0,) * len(shape))


def kernel(nodes, edges, graph, node_W0, node_b0, node_W1, node_b1,
           edge_W0, edge_b0, edge_W1, edge_b1):
    dst_flat = graph[1]
    gsrcc = graph[0].reshape(NSLICE, NW, GNCH, GCHUNK)
    gdstc = graph[1].reshape(NSLICE, NW, GNCH, GCHUNK)

    scatter_call, gather_call = _sc_calls()
    partials = scatter_call(edges, dst_flat)

    new_nodes, p_tab, q_tab = pl.pallas_call(
        _node_body,
        grid=(N // NODE_BLK,),
        in_specs=[
            pl.BlockSpec((NODE_BLK, D), lambda i: (i, 0)),
            pl.BlockSpec((NC, NODE_BLK, D), lambda i: (0, i, 0)),
            _full((2 * D, D)),
            _full((1, D)),
            _full((D, D)),
            _full((1, D)),
            _full((D, D)),
            _full((D, D)),
        ],
        out_specs=[
            pl.BlockSpec((NODE_BLK, D), lambda i: (i, 0)),
            pl.BlockSpec((NODE_BLK, D), lambda i: (i, 0)),
            pl.BlockSpec((NODE_BLK, D), lambda i: (i, 0)),
        ],
        out_shape=[
            jax.ShapeDtypeStruct((N, D), jnp.float32),
            jax.ShapeDtypeStruct((N, D), jnp.float32),
            jax.ShapeDtypeStruct((N, D), jnp.float32),
        ],
    )(nodes, partials, node_W0, node_b0.reshape(1, D), node_W1,
      node_b1.reshape(1, D), edge_W0[:D, :], edge_W0[D:2 * D, :])

    ew0c = edge_W0[2 * D:, :]
    eb0 = edge_b0.reshape(1, D)
    eb1 = edge_b1.reshape(1, D)
    blk_per_slice = ES // EDGE_BLK

    g_slices = [gather_call(p_tab, q_tab, gsrcc[i], gdstc[i])
                for i in range(NSLICE)]

    new_edges = None
    for i in range(NSLICE):
        base = i * blk_per_slice
        edge_specs = [
            pl.BlockSpec((EDGE_BLK, D), lambda j: (j, 0)),
            pl.BlockSpec((EDGE_BLK, D),
                         functools.partial(lambda b, j: (b + j, 0), base)),
            _full((D, D)),
            _full((1, D)),
            _full((D, D)),
            _full((1, D)),
        ]
        out_spec = pl.BlockSpec((EDGE_BLK, D),
                                functools.partial(lambda b, j: (b + j, 0),
                                                  base))
        if i == 0:
            new_edges = pl.pallas_call(
                _edge_body,
                grid=(blk_per_slice,),
                in_specs=edge_specs,
                out_specs=out_spec,
                out_shape=jax.ShapeDtypeStruct((E, D), jnp.float32),
            )(g_slices[i], edges, ew0c, eb0, edge_W1, eb1)
        else:
            new_edges = pl.pallas_call(
                _edge_body_alias,
                grid=(blk_per_slice,),
                in_specs=edge_specs + [pl.BlockSpec((8, D), lambda j: (0, 0))],
                out_specs=out_spec,
                out_shape=jax.ShapeDtypeStruct((E, D), jnp.float32),
                input_output_aliases={6: 0},
            )(g_slices[i], edges, ew0c, eb0, edge_W1, eb1, new_edges)

    return (new_nodes, new_edges)


# speedup vs baseline: 5.4306x; 1.0149x over previous
import functools
import jax
import jax.numpy as jnp
from jax import lax
from jax.experimental import pallas as pl
from jax.experimental.pallas import tpu as pltpu
from jax.experimental.pallas import tpu_sc as plsc

N = 10000
E = 320000
D = 128

NC = 2
NS = 16
NW = NC * NS
EW = E // NW

SCHUNK = 40
SNCH = EW // SCHUNK
NACC = N // SCHUNK
GCHUNK = 40
NSLICE = 5
ES = E // NSLICE
EWG = ES // NW
GNCH = EWG // GCHUNK
NB = 5
NBS = 5


def _scatter_body(edges_hbm, dstc_hbm, out_hbm, idx_v, ebuf, acc,
                  sem_idx, sem_in, sem_sc):
    c = lax.axis_index("c")
    s = lax.axis_index("s")
    wid = c * NS + s
    ebase = wid * EW

    @pl.loop(0, SCHUNK)
    def _zrow(i):
        @pl.loop(0, D // 16, unroll=8)
        def _zlane(l):
            ebuf[0, i, pl.ds(l * 16, 16)] = jnp.zeros((16,), jnp.float32)

    @pl.loop(s, NACC, step=NS)
    def _zacc(k):
        pltpu.sync_copy(ebuf.at[0], acc.at[pl.ds(k * SCHUNK, SCHUNK)])

    plsc.subcore_barrier()

    for b in range(NBS):
        pltpu.async_copy(dstc_hbm.at[pl.ds((wid * SNCH + b) * SCHUNK, SCHUNK)],
                         idx_v.at[b], sem_idx.at[b])
        pltpu.async_copy(edges_hbm.at[pl.ds(ebase + b * SCHUNK, SCHUNK)],
                         ebuf.at[b], sem_in.at[b])

    @pl.loop(0, SNCH // NBS)
    def _group(g):
        for b in range(NBS):
            k = g * NBS + b
            bp = (b - 1) % NBS
            pltpu.make_async_copy(
                dstc_hbm.at[pl.ds((wid * SNCH + k) * SCHUNK, SCHUNK)],
                idx_v.at[b], sem_idx.at[b]).wait()
            pltpu.make_async_copy(
                edges_hbm.at[pl.ds(ebase + k * SCHUNK, SCHUNK)],
                ebuf.at[b], sem_in.at[b]).wait()
            pltpu.async_copy(ebuf.at[b], acc.at[idx_v.at[b]], sem_sc.at[b],
                             add=True)

            @pl.when(k >= 1)
            def _drain_prev():
                pltpu.make_async_copy(
                    ebuf.at[bp], acc.at[idx_v.at[bp]], sem_sc.at[bp]).wait()

                @pl.when(k + NBS - 1 < SNCH)
                def _refill():
                    kk = k + NBS - 1
                    pltpu.async_copy(
                        dstc_hbm.at[pl.ds((wid * SNCH + kk) * SCHUNK, SCHUNK)],
                        idx_v.at[bp], sem_idx.at[bp])
                    pltpu.async_copy(
                        edges_hbm.at[pl.ds(ebase + kk * SCHUNK, SCHUNK)],
                        ebuf.at[bp], sem_in.at[bp])

    last = (SNCH - 1) % NBS
    pltpu.make_async_copy(ebuf.at[last], acc.at[idx_v.at[last]],
                          sem_sc.at[last]).wait()

    plsc.subcore_barrier()

    @pl.loop(s, NACC, step=NS)
    def _wacc(k):
        pltpu.sync_copy(acc.at[pl.ds(k * SCHUNK, SCHUNK)],
                        out_hbm.at[c, pl.ds(k * SCHUNK, SCHUNK)])


def _gather_body(p_hbm, q_hbm, srcc_hbm, dstc_hbm, g_hbm,
                 isrc, idst, pbuf, qbuf, sem_p, sem_q, sem_out):
    c = lax.axis_index("c")
    s = lax.axis_index("s")
    wid = c * NS + s
    ebase = wid * EWG

    pltpu.sync_copy(srcc_hbm.at[wid], isrc)
    pltpu.sync_copy(dstc_hbm.at[wid], idst)

    for b in range(NB):
        pltpu.async_copy(p_hbm.at[isrc.at[b]], pbuf.at[b], sem_p.at[b])
        pltpu.async_copy(q_hbm.at[idst.at[b]], qbuf.at[b], sem_q.at[b])

    @pl.loop(0, GNCH // NB)
    def _group(g):
        for b in range(NB):
            k = g * NB + b
            bp = (b - 1) % NB
            pltpu.make_async_copy(p_hbm.at[isrc.at[k]], pbuf.at[b],
                                  sem_p.at[b]).wait()
            pltpu.make_async_copy(q_hbm.at[idst.at[k]], qbuf.at[b],
                                  sem_q.at[b]).wait()

            @pl.loop(0, GCHUNK)
            def _row(i):
                @pl.loop(0, D // 16, unroll=8)
                def _lane(l):
                    sl = pl.ds(l * 16, 16)
                    pbuf[b, i, sl] = pbuf[b, i, sl] + qbuf[b, i, sl]

            pltpu.async_copy(pbuf.at[b],
                             g_hbm.at[pl.ds(ebase + k * GCHUNK, GCHUNK)],
                             sem_out.at[b])

            @pl.when(k >= 1)
            def _drain_prev():
                pltpu.make_async_copy(
                    pbuf.at[bp],
                    g_hbm.at[pl.ds(ebase + (k - 1) * GCHUNK, GCHUNK)],
                    sem_out.at[bp]).wait()

                @pl.when(k + NB - 1 < GNCH)
                def _refill():
                    kk = k + NB - 1
                    pltpu.async_copy(p_hbm.at[isrc.at[kk]], pbuf.at[bp],
                                     sem_p.at[bp])
                    pltpu.async_copy(q_hbm.at[idst.at[kk]], qbuf.at[bp],
                                     sem_q.at[bp])

    last = GNCH - 1
    pltpu.make_async_copy(pbuf.at[last % NB],
                          g_hbm.at[pl.ds(ebase + last * GCHUNK, GCHUNK)],
                          sem_out.at[last % NB]).wait()


@functools.cache
def _sc_calls():
    mesh = plsc.VectorSubcoreMesh(
        core_axis_name="c", subcore_axis_name="s",
        num_cores=NC, num_subcores=NS)
    scatter_call = pl.kernel(
        _scatter_body,
        out_type=jax.ShapeDtypeStruct((NC, N, D), jnp.float32),
        mesh=mesh,
        scratch_types=[
            pltpu.VMEM((NBS, SCHUNK), jnp.int32),
            pltpu.VMEM((NBS, SCHUNK, D), jnp.float32),
            pltpu.VMEM_SHARED((N, D), jnp.float32),
            pltpu.SemaphoreType.DMA((NBS,)),
            pltpu.SemaphoreType.DMA((NBS,)),
            pltpu.SemaphoreType.DMA((NBS,)),
        ],
    )
    gather_call = pl.kernel(
        _gather_body,
        out_type=jax.ShapeDtypeStruct((ES, D), jnp.float32),
        mesh=mesh,
        scratch_types=[
            pltpu.VMEM((GNCH, GCHUNK), jnp.int32),
            pltpu.VMEM((GNCH, GCHUNK), jnp.int32),
            pltpu.VMEM((NB, GCHUNK, D), jnp.float32),
            pltpu.VMEM((NB, GCHUNK, D), jnp.float32),
            pltpu.SemaphoreType.DMA((NB,)),
            pltpu.SemaphoreType.DMA((NB,)),
            pltpu.SemaphoreType.DMA((NB,)),
        ],
    )
    return scatter_call, gather_call


NODE_BLK = 5000
EDGE_BLK = 8000


def _node_body(nodes_ref, pr_ref, w0_ref, b0_ref, w1_ref, b1_ref,
               ew0a_ref, ew0b_ref, nn_ref, p_ref, q_ref):
    msg = pr_ref[0] + pr_ref[1]
    x = nodes_ref[...]
    h = jnp.maximum(
        jnp.dot(x, w0_ref[:D, :], preferred_element_type=jnp.float32)
        + jnp.dot(msg, w0_ref[D:, :], preferred_element_type=jnp.float32)
        + b0_ref[...], 0.0)
    nn = jnp.maximum(
        jnp.dot(h, w1_ref[...], preferred_element_type=jnp.float32)
        + b1_ref[...], 0.0) + x
    nn_ref[...] = nn
    p_ref[...] = jnp.dot(nn, ew0a_ref[...], preferred_element_type=jnp.float32)
    q_ref[...] = jnp.dot(nn, ew0b_ref[...], preferred_element_type=jnp.float32)


def _edge_body(g_ref, edges_ref, ew0c_ref, eb0_ref, ew1_ref, eb1_ref,
               out_ref):
    e = edges_ref[...]
    he = jnp.maximum(
        g_ref[...]
        + jnp.dot(e, ew0c_ref[...], preferred_element_type=jnp.float32)
        + eb0_ref[...], 0.0)
    out_ref[...] = jnp.tanh(
        jnp.dot(he, ew1_ref[...], preferred_element_type=jnp.float32)
        + eb1_ref[...]) + e


def _edge_body_alias(g_ref, edges_ref, ew0c_ref, eb0_ref, ew1_ref, eb1_ref,
                     prev_ref, out_ref):
    del prev_ref
    _edge_body(g_ref, edges_ref, ew0c_ref, eb0_ref, ew1_ref, eb1_ref, out_ref)


def _full(shape):
    return pl.BlockSpec(shape, lambda i: (0,) * len(shape))


def kernel(nodes, edges, graph, node_W0, node_b0, node_W1, node_b1,
           edge_W0, edge_b0, edge_W1, edge_b1):
    dst_flat = graph[1]
    gsrcc = graph[0].reshape(NSLICE, NW, GNCH, GCHUNK)
    gdstc = graph[1].reshape(NSLICE, NW, GNCH, GCHUNK)

    scatter_call, gather_call = _sc_calls()
    partials = scatter_call(edges, dst_flat)

    new_nodes, p_tab, q_tab = pl.pallas_call(
        _node_body,
        grid=(N // NODE_BLK,),
        in_specs=[
            pl.BlockSpec((NODE_BLK, D), lambda i: (i, 0)),
            pl.BlockSpec((NC, NODE_BLK, D), lambda i: (0, i, 0)),
            _full((2 * D, D)),
            _full((1, D)),
            _full((D, D)),
            _full((1, D)),
            _full((D, D)),
            _full((D, D)),
        ],
        out_specs=[
            pl.BlockSpec((NODE_BLK, D), lambda i: (i, 0)),
            pl.BlockSpec((NODE_BLK, D), lambda i: (i, 0)),
            pl.BlockSpec((NODE_BLK, D), lambda i: (i, 0)),
        ],
        out_shape=[
            jax.ShapeDtypeStruct((N, D), jnp.float32),
            jax.ShapeDtypeStruct((N, D), jnp.float32),
            jax.ShapeDtypeStruct((N, D), jnp.float32),
        ],
    )(nodes, partials, node_W0, node_b0.reshape(1, D), node_W1,
      node_b1.reshape(1, D), edge_W0[:D, :], edge_W0[D:2 * D, :])

    ew0c = edge_W0[2 * D:, :]
    eb0 = edge_b0.reshape(1, D)
    eb1 = edge_b1.reshape(1, D)
    blk_per_slice = ES // EDGE_BLK

    g_slices = [gather_call(p_tab, q_tab, gsrcc[i], gdstc[i])
                for i in range(NSLICE)]

    new_edges = None
    for i in range(NSLICE):
        base = i * blk_per_slice
        edge_specs = [
            pl.BlockSpec((EDGE_BLK, D), lambda j: (j, 0)),
            pl.BlockSpec((EDGE_BLK, D),
                         functools.partial(lambda b, j: (b + j, 0), base)),
            _full((D, D)),
            _full((1, D)),
            _full((D, D)),
            _full((1, D)),
        ]
        out_spec = pl.BlockSpec((EDGE_BLK, D),
                                functools.partial(lambda b, j: (b + j, 0),
                                                  base))
        if i == 0:
            new_edges = pl.pallas_call(
                _edge_body,
                grid=(blk_per_slice,),
                in_specs=edge_specs,
                out_specs=out_spec,
                out_shape=jax.ShapeDtypeStruct((E, D), jnp.float32),
            )(g_slices[i], edges, ew0c, eb0, edge_W1, eb1)
        else:
            new_edges = pl.pallas_call(
                _edge_body_alias,
                grid=(blk_per_slice,),
                in_specs=edge_specs + [pl.BlockSpec((8, D), lambda j: (0, 0))],
                out_specs=out_spec,
                out_shape=jax.ShapeDtypeStruct((E, D), jnp.float32),
                input_output_aliases={6: 0},
            )(g_slices[i], edges, ew0c, eb0, edge_W1, eb1, new_edges)

    return (new_nodes, new_edges)


# speedup vs baseline: 5.5374x; 1.0197x over previous
import functools
import jax
import jax.numpy as jnp
from jax import lax
from jax.experimental import pallas as pl
from jax.experimental.pallas import tpu as pltpu
from jax.experimental.pallas import tpu_sc as plsc

N = 10000
E = 320000
D = 128

NC = 2
NS = 16
NW = NC * NS
EW = E // NW

SCHUNK = 40
SNCH = EW // SCHUNK
NACC = N // SCHUNK
GCHUNK = 40
NSLICE = 2
ES = E // NSLICE
EWG = ES // NW
GNCH = EWG // GCHUNK
NB = 5
NBS = 5


def _scatter_body(edges_hbm, dstc_hbm, out_hbm, idx_v, ebuf, acc,
                  sem_idx, sem_in, sem_sc):
    c = lax.axis_index("c")
    s = lax.axis_index("s")
    wid = c * NS + s
    ebase = wid * EW

    @pl.loop(0, SCHUNK)
    def _zrow(i):
        @pl.loop(0, D // 16, unroll=8)
        def _zlane(l):
            ebuf[0, i, pl.ds(l * 16, 16)] = jnp.zeros((16,), jnp.float32)

    @pl.loop(s, NACC, step=NS)
    def _zacc(k):
        pltpu.sync_copy(ebuf.at[0], acc.at[pl.ds(k * SCHUNK, SCHUNK)])

    plsc.subcore_barrier()

    for b in range(NBS):
        pltpu.async_copy(dstc_hbm.at[pl.ds((wid * SNCH + b) * SCHUNK, SCHUNK)],
                         idx_v.at[b], sem_idx.at[b])
        pltpu.async_copy(edges_hbm.at[pl.ds(ebase + b * SCHUNK, SCHUNK)],
                         ebuf.at[b], sem_in.at[b])

    @pl.loop(0, SNCH // NBS)
    def _group(g):
        for b in range(NBS):
            k = g * NBS + b
            bp = (b - 1) % NBS
            pltpu.make_async_copy(
                dstc_hbm.at[pl.ds((wid * SNCH + k) * SCHUNK, SCHUNK)],
                idx_v.at[b], sem_idx.at[b]).wait()
            pltpu.make_async_copy(
                edges_hbm.at[pl.ds(ebase + k * SCHUNK, SCHUNK)],
                ebuf.at[b], sem_in.at[b]).wait()
            pltpu.async_copy(ebuf.at[b], acc.at[idx_v.at[b]], sem_sc.at[b],
                             add=True)

            @pl.when(k >= 1)
            def _drain_prev():
                pltpu.make_async_copy(
                    ebuf.at[bp], acc.at[idx_v.at[bp]], sem_sc.at[bp]).wait()

                @pl.when(k + NBS - 1 < SNCH)
                def _refill():
                    kk = k + NBS - 1
                    pltpu.async_copy(
                        dstc_hbm.at[pl.ds((wid * SNCH + kk) * SCHUNK, SCHUNK)],
                        idx_v.at[bp], sem_idx.at[bp])
                    pltpu.async_copy(
                        edges_hbm.at[pl.ds(ebase + kk * SCHUNK, SCHUNK)],
                        ebuf.at[bp], sem_in.at[bp])

    last = (SNCH - 1) % NBS
    pltpu.make_async_copy(ebuf.at[last], acc.at[idx_v.at[last]],
                          sem_sc.at[last]).wait()

    plsc.subcore_barrier()

    @pl.loop(s, NACC, step=NS)
    def _wacc(k):
        pltpu.sync_copy(acc.at[pl.ds(k * SCHUNK, SCHUNK)],
                        out_hbm.at[c, pl.ds(k * SCHUNK, SCHUNK)])


def _gather_body(p_hbm, q_hbm, srcc_hbm, dstc_hbm, g_hbm,
                 isrc, idst, pbuf, qbuf, sem_p, sem_q, sem_out):
    c = lax.axis_index("c")
    s = lax.axis_index("s")
    wid = c * NS + s
    ebase = wid * EWG

    pltpu.sync_copy(srcc_hbm.at[wid], isrc)
    pltpu.sync_copy(dstc_hbm.at[wid], idst)

    for b in range(NB):
        pltpu.async_copy(p_hbm.at[isrc.at[b]], pbuf.at[b], sem_p.at[b])
        pltpu.async_copy(q_hbm.at[idst.at[b]], qbuf.at[b], sem_q.at[b])

    @pl.loop(0, GNCH // NB)
    def _group(g):
        for b in range(NB):
            k = g * NB + b
            bp = (b - 1) % NB
            pltpu.make_async_copy(p_hbm.at[isrc.at[k]], pbuf.at[b],
                                  sem_p.at[b]).wait()
            pltpu.make_async_copy(q_hbm.at[idst.at[k]], qbuf.at[b],
                                  sem_q.at[b]).wait()

            @pl.loop(0, GCHUNK)
            def _row(i):
                @pl.loop(0, D // 16, unroll=8)
                def _lane(l):
                    sl = pl.ds(l * 16, 16)
                    pbuf[b, i, sl] = pbuf[b, i, sl] + qbuf[b, i, sl]

            pltpu.async_copy(pbuf.at[b],
                             g_hbm.at[pl.ds(ebase + k * GCHUNK, GCHUNK)],
                             sem_out.at[b])

            @pl.when(k >= 1)
            def _drain_prev():
                pltpu.make_async_copy(
                    pbuf.at[bp],
                    g_hbm.at[pl.ds(ebase + (k - 1) * GCHUNK, GCHUNK)],
                    sem_out.at[bp]).wait()

                @pl.when(k + NB - 1 < GNCH)
                def _refill():
                    kk = k + NB - 1
                    pltpu.async_copy(p_hbm.at[isrc.at[kk]], pbuf.at[bp],
                                     sem_p.at[bp])
                    pltpu.async_copy(q_hbm.at[idst.at[kk]], qbuf.at[bp],
                                     sem_q.at[bp])

    last = GNCH - 1
    pltpu.make_async_copy(pbuf.at[last % NB],
                          g_hbm.at[pl.ds(ebase + last * GCHUNK, GCHUNK)],
                          sem_out.at[last % NB]).wait()


@functools.cache
def _sc_calls():
    mesh = plsc.VectorSubcoreMesh(
        core_axis_name="c", subcore_axis_name="s",
        num_cores=NC, num_subcores=NS)
    scatter_call = pl.kernel(
        _scatter_body,
        out_type=jax.ShapeDtypeStruct((NC, N, D), jnp.float32),
        mesh=mesh,
        scratch_types=[
            pltpu.VMEM((NBS, SCHUNK), jnp.int32),
            pltpu.VMEM((NBS, SCHUNK, D), jnp.float32),
            pltpu.VMEM_SHARED((N, D), jnp.float32),
            pltpu.SemaphoreType.DMA((NBS,)),
            pltpu.SemaphoreType.DMA((NBS,)),
            pltpu.SemaphoreType.DMA((NBS,)),
        ],
    )
    gather_call = pl.kernel(
        _gather_body,
        out_type=jax.ShapeDtypeStruct((ES, D), jnp.float32),
        mesh=mesh,
        scratch_types=[
            pltpu.VMEM((GNCH, GCHUNK), jnp.int32),
            pltpu.VMEM((GNCH, GCHUNK), jnp.int32),
            pltpu.VMEM((NB, GCHUNK, D), jnp.float32),
            pltpu.VMEM((NB, GCHUNK, D), jnp.float32),
            pltpu.SemaphoreType.DMA((NB,)),
            pltpu.SemaphoreType.DMA((NB,)),
            pltpu.SemaphoreType.DMA((NB,)),
        ],
    )
    return scatter_call, gather_call


NODE_BLK = 5000
EDGE_BLK = 8000


def _node_body(nodes_ref, pr_ref, w0_ref, b0_ref, w1_ref, b1_ref,
               ew0a_ref, ew0b_ref, nn_ref, p_ref, q_ref):
    msg = pr_ref[0] + pr_ref[1]
    x = nodes_ref[...]
    h = jnp.maximum(
        jnp.dot(x, w0_ref[:D, :], preferred_element_type=jnp.float32)
        + jnp.dot(msg, w0_ref[D:, :], preferred_element_type=jnp.float32)
        + b0_ref[...], 0.0)
    nn = jnp.maximum(
        jnp.dot(h, w1_ref[...], preferred_element_type=jnp.float32)
        + b1_ref[...], 0.0) + x
    nn_ref[...] = nn
    p_ref[...] = jnp.dot(nn, ew0a_ref[...], preferred_element_type=jnp.float32)
    q_ref[...] = jnp.dot(nn, ew0b_ref[...], preferred_element_type=jnp.float32)


def _edge_body(g_ref, edges_ref, ew0c_ref, eb0_ref, ew1_ref, eb1_ref,
               out_ref):
    e = edges_ref[...]
    he = jnp.maximum(
        g_ref[...]
        + jnp.dot(e, ew0c_ref[...], preferred_element_type=jnp.float32)
        + eb0_ref[...], 0.0)
    out_ref[...] = jnp.tanh(
        jnp.dot(he, ew1_ref[...], preferred_element_type=jnp.float32)
        + eb1_ref[...]) + e


def _edge_body_alias(g_ref, edges_ref, ew0c_ref, eb0_ref, ew1_ref, eb1_ref,
                     prev_ref, out_ref):
    del prev_ref
    _edge_body(g_ref, edges_ref, ew0c_ref, eb0_ref, ew1_ref, eb1_ref, out_ref)


def _full(shape):
    return pl.BlockSpec(shape, lambda i: (0,) * len(shape))


def kernel(nodes, edges, graph, node_W0, node_b0, node_W1, node_b1,
           edge_W0, edge_b0, edge_W1, edge_b1):
    dst_flat = graph[1]
    gsrcc = graph[0].reshape(NSLICE, NW, GNCH, GCHUNK)
    gdstc = graph[1].reshape(NSLICE, NW, GNCH, GCHUNK)

    scatter_call, gather_call = _sc_calls()
    partials = scatter_call(edges, dst_flat)

    new_nodes, p_tab, q_tab = pl.pallas_call(
        _node_body,
        grid=(N // NODE_BLK,),
        in_specs=[
            pl.BlockSpec((NODE_BLK, D), lambda i: (i, 0)),
            pl.BlockSpec((NC, NODE_BLK, D), lambda i: (0, i, 0)),
            _full((2 * D, D)),
            _full((1, D)),
            _full((D, D)),
            _full((1, D)),
            _full((D, D)),
            _full((D, D)),
        ],
        out_specs=[
            pl.BlockSpec((NODE_BLK, D), lambda i: (i, 0)),
            pl.BlockSpec((NODE_BLK, D), lambda i: (i, 0)),
            pl.BlockSpec((NODE_BLK, D), lambda i: (i, 0)),
        ],
        out_shape=[
            jax.ShapeDtypeStruct((N, D), jnp.float32),
            jax.ShapeDtypeStruct((N, D), jnp.float32),
            jax.ShapeDtypeStruct((N, D), jnp.float32),
        ],
    )(nodes, partials, node_W0, node_b0.reshape(1, D), node_W1,
      node_b1.reshape(1, D), edge_W0[:D, :], edge_W0[D:2 * D, :])

    ew0c = edge_W0[2 * D:, :]
    eb0 = edge_b0.reshape(1, D)
    eb1 = edge_b1.reshape(1, D)
    blk_per_slice = ES // EDGE_BLK

    g_slices = [gather_call(p_tab, q_tab, gsrcc[i], gdstc[i])
                for i in range(NSLICE)]

    new_edges = None
    for i in range(NSLICE):
        base = i * blk_per_slice
        edge_specs = [
            pl.BlockSpec((EDGE_BLK, D), lambda j: (j, 0)),
            pl.BlockSpec((EDGE_BLK, D),
                         functools.partial(lambda b, j: (b + j, 0), base)),
            _full((D, D)),
            _full((1, D)),
            _full((D, D)),
            _full((1, D)),
        ]
        out_spec = pl.BlockSpec((EDGE_BLK, D),
                                functools.partial(lambda b, j: (b + j, 0),
                                                  base))
        if i == 0:
            new_edges = pl.pallas_call(
                _edge_body,
                grid=(blk_per_slice,),
                in_specs=edge_specs,
                out_specs=out_spec,
                out_shape=jax.ShapeDtypeStruct((E, D), jnp.float32),
            )(g_slices[i], edges, ew0c, eb0, edge_W1, eb1)
        else:
            new_edges = pl.pallas_call(
                _edge_body_alias,
                grid=(blk_per_slice,),
                in_specs=edge_specs + [pl.BlockSpec((8, D), lambda j: (0, 0))],
                out_specs=out_spec,
                out_shape=jax.ShapeDtypeStruct((E, D), jnp.float32),
                input_output_aliases={6: 0},
            )(g_slices[i], edges, ew0c, eb0, edge_W1, eb1, new_edges)

    return (new_nodes, new_edges)


# speedup vs baseline: 5.5662x; 1.0052x over previous
import functools
import jax
import jax.numpy as jnp
from jax import lax
from jax.experimental import pallas as pl
from jax.experimental.pallas import tpu as pltpu
from jax.experimental.pallas import tpu_sc as plsc

N = 10000
E = 320000
D = 128

NC = 2
NS = 16
NW = NC * NS
EW = E // NW

SCHUNK = 40
SNCH = EW // SCHUNK
NACC = N // SCHUNK
GCHUNK = 40
NSLICE = 2
ES = E // NSLICE
EWG = ES // NW
GNCH = EWG // GCHUNK
NB = 5
NBS = 5


def _scatter_body(edges_hbm, dstc_hbm, out_hbm, idx_v, ebuf, acc,
                  sem_idx, sem_in, sem_sc):
    c = lax.axis_index("c")
    s = lax.axis_index("s")
    wid = c * NS + s
    ebase = wid * EW

    @pl.loop(0, SCHUNK)
    def _zrow(i):
        @pl.loop(0, D // 16, unroll=8)
        def _zlane(l):
            ebuf[0, i, pl.ds(l * 16, 16)] = jnp.zeros((16,), jnp.float32)

    @pl.loop(s, NACC, step=NS)
    def _zacc(k):
        pltpu.sync_copy(ebuf.at[0], acc.at[pl.ds(k * SCHUNK, SCHUNK)])

    plsc.subcore_barrier()

    for b in range(NBS):
        pltpu.async_copy(dstc_hbm.at[pl.ds((wid * SNCH + b) * SCHUNK, SCHUNK)],
                         idx_v.at[b], sem_idx.at[b])
        pltpu.async_copy(edges_hbm.at[pl.ds(ebase + b * SCHUNK, SCHUNK)],
                         ebuf.at[b], sem_in.at[b])

    @pl.loop(0, SNCH // NBS)
    def _group(g):
        for b in range(NBS):
            k = g * NBS + b
            bp = (b - 1) % NBS
            pltpu.make_async_copy(
                dstc_hbm.at[pl.ds((wid * SNCH + k) * SCHUNK, SCHUNK)],
                idx_v.at[b], sem_idx.at[b]).wait()
            pltpu.make_async_copy(
                edges_hbm.at[pl.ds(ebase + k * SCHUNK, SCHUNK)],
                ebuf.at[b], sem_in.at[b]).wait()
            pltpu.async_copy(ebuf.at[b], acc.at[idx_v.at[b]], sem_sc.at[b],
                             add=True)

            @pl.when(k >= 1)
            def _drain_prev():
                pltpu.make_async_copy(
                    ebuf.at[bp], acc.at[idx_v.at[bp]], sem_sc.at[bp]).wait()

                @pl.when(k + NBS - 1 < SNCH)
                def _refill():
                    kk = k + NBS - 1
                    pltpu.async_copy(
                        dstc_hbm.at[pl.ds((wid * SNCH + kk) * SCHUNK, SCHUNK)],
                        idx_v.at[bp], sem_idx.at[bp])
                    pltpu.async_copy(
                        edges_hbm.at[pl.ds(ebase + kk * SCHUNK, SCHUNK)],
                        ebuf.at[bp], sem_in.at[bp])

    last = (SNCH - 1) % NBS
    pltpu.make_async_copy(ebuf.at[last], acc.at[idx_v.at[last]],
                          sem_sc.at[last]).wait()

    plsc.subcore_barrier()

    @pl.loop(s, NACC, step=NS)
    def _wacc(k):
        pltpu.sync_copy(acc.at[pl.ds(k * SCHUNK, SCHUNK)],
                        out_hbm.at[c, pl.ds(k * SCHUNK, SCHUNK)])


def _gather_body(p_hbm, q_hbm, srcc_hbm, dstc_hbm, g_hbm,
                 isrc, idst, pbuf, qbuf, sem_p, sem_q, sem_out):
    c = lax.axis_index("c")
    s = lax.axis_index("s")
    wid = c * NS + s
    ebase = wid * EWG

    pltpu.sync_copy(srcc_hbm.at[wid], isrc)
    pltpu.sync_copy(dstc_hbm.at[wid], idst)

    for b in range(NB):
        pltpu.async_copy(p_hbm.at[isrc.at[b]], pbuf.at[b], sem_p.at[b])
        pltpu.async_copy(q_hbm.at[idst.at[b]], qbuf.at[b], sem_q.at[b])

    @pl.loop(0, GNCH // NB)
    def _group(g):
        for b in range(NB):
            k = g * NB + b
            bp = (b - 1) % NB
            pltpu.make_async_copy(p_hbm.at[isrc.at[k]], pbuf.at[b],
                                  sem_p.at[b]).wait()
            pltpu.make_async_copy(q_hbm.at[idst.at[k]], qbuf.at[b],
                                  sem_q.at[b]).wait()

            @pl.loop(0, GCHUNK)
            def _row(i):
                @pl.loop(0, D // 16, unroll=8)
                def _lane(l):
                    sl = pl.ds(l * 16, 16)
                    pbuf[b, i, sl] = pbuf[b, i, sl] + qbuf[b, i, sl]

            pltpu.async_copy(pbuf.at[b],
                             g_hbm.at[pl.ds(ebase + k * GCHUNK, GCHUNK)],
                             sem_out.at[b])

            @pl.when(k >= 1)
            def _drain_prev():
                pltpu.make_async_copy(
                    pbuf.at[bp],
                    g_hbm.at[pl.ds(ebase + (k - 1) * GCHUNK, GCHUNK)],
                    sem_out.at[bp]).wait()

                @pl.when(k + NB - 1 < GNCH)
                def _refill():
                    kk = k + NB - 1
                    pltpu.async_copy(p_hbm.at[isrc.at[kk]], pbuf.at[bp],
                                     sem_p.at[bp])
                    pltpu.async_copy(q_hbm.at[idst.at[kk]], qbuf.at[bp],
                                     sem_q.at[bp])

    last = GNCH - 1
    pltpu.make_async_copy(pbuf.at[last % NB],
                          g_hbm.at[pl.ds(ebase + last * GCHUNK, GCHUNK)],
                          sem_out.at[last % NB]).wait()


@functools.cache
def _sc_calls():
    mesh = plsc.VectorSubcoreMesh(
        core_axis_name="c", subcore_axis_name="s",
        num_cores=NC, num_subcores=NS)
    scatter_call = pl.kernel(
        _scatter_body,
        out_type=jax.ShapeDtypeStruct((NC, N, D), jnp.float32),
        mesh=mesh,
        scratch_types=[
            pltpu.VMEM((NBS, SCHUNK), jnp.int32),
            pltpu.VMEM((NBS, SCHUNK, D), jnp.float32),
            pltpu.VMEM_SHARED((N, D), jnp.float32),
            pltpu.SemaphoreType.DMA((NBS,)),
            pltpu.SemaphoreType.DMA((NBS,)),
            pltpu.SemaphoreType.DMA((NBS,)),
        ],
    )
    gather_call = pl.kernel(
        _gather_body,
        out_type=jax.ShapeDtypeStruct((ES, D), jnp.float32),
        mesh=mesh,
        scratch_types=[
            pltpu.VMEM((GNCH, GCHUNK), jnp.int32),
            pltpu.VMEM((GNCH, GCHUNK), jnp.int32),
            pltpu.VMEM((NB, GCHUNK, D), jnp.float32),
            pltpu.VMEM((NB, GCHUNK, D), jnp.float32),
            pltpu.SemaphoreType.DMA((NB,)),
            pltpu.SemaphoreType.DMA((NB,)),
            pltpu.SemaphoreType.DMA((NB,)),
        ],
    )
    return scatter_call, gather_call


NODE_BLK = 5000
EDGE_BLK = 16000


def _node_body(nodes_ref, pr_ref, w0_ref, b0_ref, w1_ref, b1_ref,
               ew0a_ref, ew0b_ref, nn_ref, p_ref, q_ref):
    msg = pr_ref[0] + pr_ref[1]
    x = nodes_ref[...]
    h = jnp.maximum(
        jnp.dot(x, w0_ref[:D, :], preferred_element_type=jnp.float32)
        + jnp.dot(msg, w0_ref[D:, :], preferred_element_type=jnp.float32)
        + b0_ref[...], 0.0)
    nn = jnp.maximum(
        jnp.dot(h, w1_ref[...], preferred_element_type=jnp.float32)
        + b1_ref[...], 0.0) + x
    nn_ref[...] = nn
    p_ref[...] = jnp.dot(nn, ew0a_ref[...], preferred_element_type=jnp.float32)
    q_ref[...] = jnp.dot(nn, ew0b_ref[...], preferred_element_type=jnp.float32)


def _edge_body(g_ref, edges_ref, ew0c_ref, eb0_ref, ew1_ref, eb1_ref,
               out_ref):
    e = edges_ref[...]
    he = jnp.maximum(
        g_ref[...]
        + jnp.dot(e, ew0c_ref[...], preferred_element_type=jnp.float32)
        + eb0_ref[...], 0.0)
    out_ref[...] = jnp.tanh(
        jnp.dot(he, ew1_ref[...], preferred_element_type=jnp.float32)
        + eb1_ref[...]) + e


def _edge_body_alias(g_ref, edges_ref, ew0c_ref, eb0_ref, ew1_ref, eb1_ref,
                     prev_ref, out_ref):
    del prev_ref
    _edge_body(g_ref, edges_ref, ew0c_ref, eb0_ref, ew1_ref, eb1_ref, out_ref)


def _full(shape):
    return pl.BlockSpec(shape, lambda i: (0,) * len(shape))


def kernel(nodes, edges, graph, node_W0, node_b0, node_W1, node_b1,
           edge_W0, edge_b0, edge_W1, edge_b1):
    dst_flat = graph[1]
    gsrcc = graph[0].reshape(NSLICE, NW, GNCH, GCHUNK)
    gdstc = graph[1].reshape(NSLICE, NW, GNCH, GCHUNK)

    scatter_call, gather_call = _sc_calls()
    partials = scatter_call(edges, dst_flat)

    new_nodes, p_tab, q_tab = pl.pallas_call(
        _node_body,
        grid=(N // NODE_BLK,),
        in_specs=[
            pl.BlockSpec((NODE_BLK, D), lambda i: (i, 0)),
            pl.BlockSpec((NC, NODE_BLK, D), lambda i: (0, i, 0)),
            _full((2 * D, D)),
            _full((1, D)),
            _full((D, D)),
            _full((1, D)),
            _full((D, D)),
            _full((D, D)),
        ],
        out_specs=[
            pl.BlockSpec((NODE_BLK, D), lambda i: (i, 0)),
            pl.BlockSpec((NODE_BLK, D), lambda i: (i, 0)),
            pl.BlockSpec((NODE_BLK, D), lambda i: (i, 0)),
        ],
        out_shape=[
            jax.ShapeDtypeStruct((N, D), jnp.float32),
            jax.ShapeDtypeStruct((N, D), jnp.float32),
            jax.ShapeDtypeStruct((N, D), jnp.float32),
        ],
    )(nodes, partials, node_W0, node_b0.reshape(1, D), node_W1,
      node_b1.reshape(1, D), edge_W0[:D, :], edge_W0[D:2 * D, :])

    ew0c = edge_W0[2 * D:, :]
    eb0 = edge_b0.reshape(1, D)
    eb1 = edge_b1.reshape(1, D)
    blk_per_slice = ES // EDGE_BLK

    g_slices = [gather_call(p_tab, q_tab, gsrcc[i], gdstc[i])
                for i in range(NSLICE)]

    new_edges = None
    for i in range(NSLICE):
        base = i * blk_per_slice
        edge_specs = [
            pl.BlockSpec((EDGE_BLK, D), lambda j: (j, 0)),
            pl.BlockSpec((EDGE_BLK, D),
                         functools.partial(lambda b, j: (b + j, 0), base)),
            _full((D, D)),
            _full((1, D)),
            _full((D, D)),
            _full((1, D)),
        ]
        out_spec = pl.BlockSpec((EDGE_BLK, D),
                                functools.partial(lambda b, j: (b + j, 0),
                                                  base))
        if i == 0:
            new_edges = pl.pallas_call(
                _edge_body,
                grid=(blk_per_slice,),
                in_specs=edge_specs,
                out_specs=out_spec,
                out_shape=jax.ShapeDtypeStruct((E, D), jnp.float32),
            )(g_slices[i], edges, ew0c, eb0, edge_W1, eb1)
        else:
            new_edges = pl.pallas_call(
                _edge_body_alias,
                grid=(blk_per_slice,),
                in_specs=edge_specs + [pl.BlockSpec((8, D), lambda j: (0, 0))],
                out_specs=out_spec,
                out_shape=jax.ShapeDtypeStruct((E, D), jnp.float32),
                input_output_aliases={6: 0},
            )(g_slices[i], edges, ew0c, eb0, edge_W1, eb1, new_edges)

    return (new_nodes, new_edges)
